# Initial kernel scaffold; baseline (speedup 1.0000x reference)
import jax
import jax.numpy as jnp
from jax.experimental import pallas as pl


def kernel(x, edge_index, edge_attr, edge_index_test, W_gat, att_src, att_dst, b_gat, bn1_g, bn1_b, bn2_g, bn2_b, W_c, b_c, W_mu, b_mu, W_ls, b_ls, W_l0, b_l0, W_l1, b_l1, W_l2, b_l2):
    raise NotImplementedError("write your pallas kernel here")


# speedup vs baseline: 6.7708x; 6.7708x over previous
import functools

import jax
import jax.numpy as jnp
from jax import lax
from jax.experimental import pallas as pl
from jax.experimental.pallas import tpu as pltpu
from jax.experimental.pallas import tpu_sc as plsc

N = 10000
NP = 10240
IN = 256
HID = 128
H = 4
NE = 5
LH = 256
E = 160000

NC, NS, L = 2, 16, 16
NW = NC * NS
CH = 128

EG = 172032
EW = 163840
ET = 163840
RPT = NP // NS

_f32 = jnp.float32
_i32 = jnp.int32


def _mesh():
    return plsc.VectorSubcoreMesh(core_axis_name="c", subcore_axis_name="s",
                                  num_cores=NC, num_subcores=NS)


def _k1a_body(src_r, dst_r, als_r, ald_r, e_out, si, di, av, bv, e16, sem):
    cid = lax.axis_index("c")
    sid = lax.axis_index("s")
    w = sid * NC + cid
    nch = EG // (NW * CH)

    def chunk(ch, _):
        base = w * (EG // NW) + ch * CH
        pltpu.sync_copy(src_r.at[pl.ds(base, CH)], si)
        pltpu.sync_copy(dst_r.at[pl.ds(base, CH)], di)
        pltpu.async_copy(als_r.at[si], av, sem).wait()
        pltpu.async_copy(ald_r.at[di], bv, sem).wait()

        def row(r, c):
            v = av[r, pl.ds(0, L)] + bv[r, pl.ds(0, L)]
            v = jnp.where(v >= 0, v, 0.2 * v)
            e16[pl.ds(r * L, L)] = jnp.exp(v)
            return c
        lax.fori_loop(0, CH, row, 0)
        pltpu.sync_copy(e16, e_out.at[pl.ds(base * L, CH * L)])
        return _
    lax.fori_loop(0, nch, chunk, 0)


def _k1a_call(srcg, dstg, als128, ald128):
    f = pl.kernel(
        _k1a_body,
        out_type=jax.ShapeDtypeStruct((EG * L,), _f32),
        mesh=_mesh(),
        scratch_types=[
            pltpu.VMEM((CH,), _i32), pltpu.VMEM((CH,), _i32),
            pltpu.VMEM((CH, HID), _f32), pltpu.VMEM((CH, HID), _f32),
            pltpu.VMEM((CH * L,), _f32),
            pltpu.SemaphoreType.DMA,
        ],
    )
    return f(srcg, dstg, als128, ald128)


def _k2a_body(dst_r, w_r, z_r, d_out, di, wv, wv128, dacc, *, etot):
    cid = lax.axis_index("c")
    sid = lax.axis_index("s")
    w = sid * NC + cid
    rows0 = sid * RPT
    pltpu.sync_copy(z_r.at[pl.ds(rows0, RPT)], dacc.at[pl.ds(rows0, RPT)])

    def zrow(r, c):
        for j in range(1, HID // L):
            wv128[r, pl.ds(j * L, L)] = jnp.zeros((L,), _f32)
        return c
    lax.fori_loop(0, CH, zrow, 0)
    plsc.subcore_barrier()
    nch = etot // (NW * CH)

    def chunk(ch, _):
        base = w * (etot // NW) + ch * CH
        pltpu.sync_copy(dst_r.at[pl.ds(base, CH)], di)
        pltpu.sync_copy(w_r.at[pl.ds(base * L, CH * L)], wv)

        def row(r, c):
            wv128[r, pl.ds(0, L)] = wv[pl.ds(r * L, L)]
            return c
        lax.fori_loop(0, CH, row, 0)
        pltpu.sync_copy(wv128, dacc.at[di], add=True)
        return _
    lax.fori_loop(0, nch, chunk, 0)
    plsc.subcore_barrier()
    pltpu.sync_copy(dacc.at[pl.ds(rows0, RPT)],
                    d_out.at[pl.ds(cid * NP + rows0, RPT)])


def _k2a_call(dst, wflat, z128, etot):
    body = functools.partial(_k2a_body, etot=etot)
    f = pl.kernel(
        body,
        out_type=jax.ShapeDtypeStruct((NC * NP, HID), _f32),
        mesh=_mesh(),
        scratch_types=[
            pltpu.VMEM((CH,), _i32), pltpu.VMEM((CH * L,), _f32),
            pltpu.VMEM((CH, HID), _f32),
            pltpu.VMEM_SHARED((NP, HID), _f32),
        ],
    )
    return f(dst, wflat, z128)


def _k2b_body(src_r, w_r, dinv_r, s_out, si, wv, dv, sem):
    cid = lax.axis_index("c")
    sid = lax.axis_index("s")
    w = sid * NC + cid
    nch = EW // (NW * CH)

    def chunk(ch, _):
        base = w * (EW // NW) + ch * CH
        pltpu.sync_copy(src_r.at[pl.ds(base, CH)], si)
        pltpu.sync_copy(w_r.at[pl.ds(base * L, CH * L)], wv)
        pltpu.async_copy(dinv_r.at[si], dv, sem).wait()

        def row(r, c):
            wv[pl.ds(r * L, L)] = wv[pl.ds(r * L, L)] * dv[r, pl.ds(0, L)]
            return c
        lax.fori_loop(0, CH, row, 0)
        pltpu.sync_copy(wv, s_out.at[pl.ds(base * L, CH * L)])
        return _
    lax.fori_loop(0, nch, chunk, 0)


def _k2b_call(srcw, wflat, dinv128):
    f = pl.kernel(
        _k2b_body,
        out_type=jax.ShapeDtypeStruct((EW * L,), _f32),
        mesh=_mesh(),
        scratch_types=[
            pltpu.VMEM((CH,), _i32), pltpu.VMEM((CH * L,), _f32),
            pltpu.VMEM((CH, HID), _f32), pltpu.SemaphoreType.DMA,
        ],
    )
    return f(srcw, wflat, dinv128)


def _k3_body(src_r, dst_r, sc_r, tab_r, z_r, out_r,
             si, di, gi, sv, rows, acc, sem, *, nbp, ncol, etot):
    cid = lax.axis_index("c")
    sid = lax.axis_index("s")
    rows0 = sid * RPT
    ept = etot // NS
    nch = ept // CH
    def block(k, carry):
        b = k * NC + cid
        boff = b * NP
        colv = jnp.full((L,), lax.rem(b, jnp.int32(ncol)), _i32)
        pltpu.sync_copy(z_r.at[pl.ds(rows0, RPT)], acc.at[pl.ds(rows0, RPT)])
        plsc.subcore_barrier()

        def chunk(ch, _):
            base = sid * ept + ch * CH
            pltpu.sync_copy(src_r.at[pl.ds(base, CH)], si)
            pltpu.sync_copy(dst_r.at[pl.ds(base, CH)], di)
            pltpu.sync_copy(sc_r.at[pl.ds(base * L, CH * L)], sv)

            def addi(r, c):
                gi[pl.ds(r * L, L)] = si[pl.ds(r * L, L)] + boff
                return c
            lax.fori_loop(0, CH // L, addi, 0)
            pltpu.async_copy(tab_r.at[gi], rows, sem).wait()

            def edge(e, c):
                v = sv[pl.ds(e * L, L)]
                sp = v[colv]
                for j in range(HID // L):
                    rows[e, pl.ds(j * L, L)] = rows[e, pl.ds(j * L, L)] * sp
                return c
            lax.fori_loop(0, CH, edge, 0)
            pltpu.sync_copy(rows, acc.at[di], add=True)
            return _
        lax.fori_loop(0, nch, chunk, 0)
        plsc.subcore_barrier()
        pltpu.sync_copy(acc.at[pl.ds(rows0, RPT)],
                        out_r.at[pl.ds(boff + rows0, RPT)])
        plsc.subcore_barrier()
        return carry
    lax.fori_loop(0, nbp // NC, block, 0)


def _k3_call(src, dst, scale, table, z128, *, nbp, ncol, etot):
    body = functools.partial(_k3_body, nbp=nbp, ncol=ncol, etot=etot)
    f = pl.kernel(
        body,
        out_type=jax.ShapeDtypeStruct((nbp * NP, HID), _f32),
        mesh=_mesh(),
        scratch_types=[
            pltpu.VMEM((CH,), _i32), pltpu.VMEM((CH,), _i32),
            pltpu.VMEM((CH,), _i32), pltpu.VMEM((CH * L,), _f32),
            pltpu.VMEM((CH, HID), _f32),
            pltpu.VMEM_SHARED((NP, HID), _f32),
            pltpu.SemaphoreType.DMA,
        ],
    )
    return f(src, dst, scale, table, z128)


def _k4_body(st_r, dt_r, pq_r, r00, r01, r10, r11, si, di, gi, av, bv, sem):
    cid = lax.axis_index("c")
    sid = lax.axis_index("s")
    w = sid * NC + cid
    nch = ET // (NW * CH)

    def chunk(ch, _):
        base = w * (ET // NW) + ch * CH
        pltpu.sync_copy(st_r.at[pl.ds(base, CH)], si)
        pltpu.sync_copy(dt_r.at[pl.ds(base, CH)], di)
        pltpu.async_copy(pq_r.at[si], av, sem).wait()
        pltpu.sync_copy(av, r00.at[pl.ds(base, CH)])

        def addi_d(r, c):
            gi[pl.ds(r * L, L)] = di[pl.ds(r * L, L)] + NP
            return c
        lax.fori_loop(0, CH // L, addi_d, 0)
        pltpu.async_copy(pq_r.at[gi], bv, sem).wait()
        pltpu.sync_copy(bv, r01.at[pl.ds(base, CH)])

        pltpu.async_copy(pq_r.at[di], av, sem).wait()
        pltpu.sync_copy(av, r10.at[pl.ds(base, CH)])

        def addi_s(r, c):
            gi[pl.ds(r * L, L)] = si[pl.ds(r * L, L)] + NP
            return c
        lax.fori_loop(0, CH // L, addi_s, 0)
        pltpu.async_copy(pq_r.at[gi], bv, sem).wait()
        pltpu.sync_copy(bv, r11.at[pl.ds(base, CH)])
        return _
    lax.fori_loop(0, nch, chunk, 0)


def _k4_call(st, dt, pq):
    o = jax.ShapeDtypeStruct((ET, LH), _f32)
    f = pl.kernel(
        _k4_body,
        out_type=(o, o, o, o),
        mesh=_mesh(),
        scratch_types=[
            pltpu.VMEM((CH,), _i32), pltpu.VMEM((CH,), _i32),
            pltpu.VMEM((CH,), _i32),
            pltpu.VMEM((CH, LH), _f32), pltpu.VMEM((CH, LH), _f32),
            pltpu.SemaphoreType.DMA,
        ],
    )
    return f(st, dt, pq)


RB = 256
NRB = NP // RB


def _m1_body(xp, wg, ats, atd, xlbm, als128, ald128):
    a = jnp.dot(xp[...], wg[...], preferred_element_type=_f32)
    a3 = a.reshape(RB, H, HID)
    xlbm[...] = a3.swapaxes(0, 1)
    als = (a3 * ats[...][None]).sum(-1)
    ald = (a3 * atd[...][None]).sum(-1)
    z = jnp.zeros((RB, HID - H), _f32)
    als128[...] = jnp.concatenate([als, z], axis=1)
    ald128[...] = jnp.concatenate([ald, z], axis=1)


def _m1_call(xp, W_gat, att_src, att_dst):
    return pl.pallas_call(
        _m1_body,
        grid=(NRB,),
        in_specs=[
            pl.BlockSpec((RB, IN), lambda i: (i, 0)),
            pl.BlockSpec((IN, H * HID), lambda i: (0, 0)),
            pl.BlockSpec((H, HID), lambda i: (0, 0)),
            pl.BlockSpec((H, HID), lambda i: (0, 0)),
        ],
        out_specs=[
            pl.BlockSpec((H, RB, HID), lambda i: (0, i, 0)),
            pl.BlockSpec((RB, HID), lambda i: (i, 0)),
            pl.BlockSpec((RB, HID), lambda i: (i, 0)),
        ],
        out_shape=[
            jax.ShapeDtypeStruct((H, NP, HID), _f32),
            jax.ShapeDtypeStruct((NP, HID), _f32),
            jax.ShapeDtypeStruct((NP, HID), _f32),
        ],
    )(xp, W_gat, att_src, att_dst)


def _m2b_body(degp, dinv128):
    d = degp[0] + degp[1] + 1.0
    dinv128[...] = lax.rsqrt(d)


def _m2b_call(degp):
    return pl.pallas_call(
        _m2b_body,
        grid=(NRB,),
        in_specs=[pl.BlockSpec((NC, RB, HID), lambda i: (0, i, 0))],
        out_specs=pl.BlockSpec((RB, HID), lambda i: (i, 0)),
        out_shape=jax.ShapeDtypeStruct((NP, HID), _f32),
    )(degp)


def _rowmask(i):
    rows = i * RB + lax.broadcasted_iota(_i32, (RB, 1), 0)
    return rows < N


def _m2_body(agg, sp, bg, g, b, out, acc):
    p = pl.program_id(0)
    i = pl.program_id(1)
    s = sp[0] + sp[1]
    s4 = s[:, :H]
    at = agg[...].swapaxes(0, 1)
    y3 = at / (s4[:, :, None] + 1e-16)
    y = y3.reshape(RB, H * HID) + bg[0]

    @pl.when(p == 0)
    def _():
        @pl.when(i == 0)
        def _():
            acc[...] = jnp.zeros_like(acc)
        acc[0] = acc[0] + y.sum(0)
        acc[1] = acc[1] + (y * y).sum(0)

    @pl.when(p == 1)
    def _():
        mean = acc[0] / N
        var = acc[1] / N - mean * mean
        yn = (y - mean) * lax.rsqrt(var + 1e-5) * g[0] + b[0]
        out[...] = jnp.where(_rowmask(i), jnp.maximum(yn, 0.0), 0.0)


def _m2_call(agg, sparts, b_gat, bn1_g, bn1_b):
    D = H * HID
    return pl.pallas_call(
        _m2_body,
        grid=(2, NRB),
        in_specs=[
            pl.BlockSpec((H, RB, HID), lambda p, i: (0, i, 0)),
            pl.BlockSpec((NC, RB, HID), lambda p, i: (0, i, 0)),
            pl.BlockSpec((1, D), lambda p, i: (0, 0)),
            pl.BlockSpec((1, D), lambda p, i: (0, 0)),
            pl.BlockSpec((1, D), lambda p, i: (0, 0)),
        ],
        out_specs=pl.BlockSpec((RB, D), lambda p, i: (i, 0)),
        out_shape=jax.ShapeDtypeStruct((NP, D), _f32),
        scratch_shapes=[pltpu.VMEM((8, D), _f32)],
    )(agg, sparts, b_gat, bn1_g, bn1_b)


def _mm_body(h, w, out):
    out[...] = jnp.dot(h[...], w[0], preferred_element_type=_f32)[None]


def _mm_call(h, wstack, nb, din):
    return pl.pallas_call(
        _mm_body,
        grid=(nb, NRB),
        in_specs=[
            pl.BlockSpec((RB, din), lambda j, r: (r, 0)),
            pl.BlockSpec((1, din, HID), lambda j, r: (j, 0, 0)),
        ],
        out_specs=pl.BlockSpec((1, RB, HID), lambda j, r: (j, r, 0)),
        out_shape=jax.ShapeDtypeStruct((nb, NP, HID), _f32),
    )(h, wstack)


def _m4_body(agg, hp, dinv, bc, g, b, out, acc):
    p = pl.program_id(0)
    i = pl.program_id(1)
    dv = dinv[:, :NE]
    at = agg[...].swapaxes(0, 1)
    ht = hp[...].swapaxes(0, 1)
    y3 = at * dv[:, :, None] + ht * (dv * dv)[:, :, None] + bc[...][None]
    y = y3.reshape(RB, NE * HID)

    @pl.when(p == 0)
    def _():
        @pl.when(i == 0)
        def _():
            acc[...] = jnp.zeros_like(acc)
        acc[0] = acc[0] + y.sum(0)
        acc[1] = acc[1] + (y * y).sum(0)

    @pl.when(p == 1)
    def _():
        mean = acc[0] / N
        var = acc[1] / N - mean * mean
        yn = (y - mean) * lax.rsqrt(var + 1e-5) * g[0] + b[0]
        out[...] = jnp.where(_rowmask(i), jnp.maximum(yn, 0.0), 0.0)


def _m4_call(agg, hp, dinv16, b_c, bn2_g, bn2_b):
    D = NE * HID
    return pl.pallas_call(
        _m4_body,
        grid=(2, NRB),
        in_specs=[
            pl.BlockSpec((NE, RB, HID), lambda p, i: (0, i, 0)),
            pl.BlockSpec((NE, RB, HID), lambda p, i: (0, i, 0)),
            pl.BlockSpec((RB, HID), lambda p, i: (i, 0)),
            pl.BlockSpec((NE, HID), lambda p, i: (0, 0)),
            pl.BlockSpec((1, D), lambda p, i: (0, 0)),
            pl.BlockSpec((1, D), lambda p, i: (0, 0)),
        ],
        out_specs=pl.BlockSpec((RB, D), lambda p, i: (i, 0)),
        out_shape=jax.ShapeDtypeStruct((NP, D), _f32),
        scratch_shapes=[pltpu.VMEM((8, D), _f32)],
    )(agg, hp, dinv16, b_c, bn2_g, bn2_b)


def _m6_body(agg, hp, dinv, bml, g, b, omu, ols, acc):
    p = pl.program_id(0)
    i = pl.program_id(1)
    dv = dinv[:, :NE]
    dvv = jnp.concatenate([dv, dv], axis=1)
    at = agg[...].swapaxes(0, 1)
    ht = hp[...].swapaxes(0, 1)
    y3 = at * dvv[:, :, None] + ht * (dvv * dvv)[:, :, None] + bml[...][None]
    D = NE * HID
    ymu = y3[:, :NE].reshape(RB, D)
    yls = y3[:, NE:].reshape(RB, D)

    @pl.when(p == 0)
    def _():
        @pl.when(i == 0)
        def _():
            acc[...] = jnp.zeros_like(acc)
        acc[0] = acc[0] + ymu.sum(0)
        acc[1] = acc[1] + (ymu * ymu).sum(0)
        acc[2] = acc[2] + yls.sum(0)
        acc[3] = acc[3] + (yls * yls).sum(0)

    @pl.when(p == 1)
    def _():
        m = _rowmask(i)
        mu_m = acc[0] / N
        mu_v = acc[1] / N - mu_m * mu_m
        ls_m = acc[2] / N
        ls_v = acc[3] / N - ls_m * ls_m
        a = (ymu - mu_m) * lax.rsqrt(mu_v + 1e-5) * g[0] + b[0]
        c = (yls - ls_m) * lax.rsqrt(ls_v + 1e-5) * g[0] + b[0]
        omu[...] = jnp.where(m, jnp.maximum(a, 0.0), 0.0)
        ols[...] = jnp.where(m, jnp.maximum(c, 0.0), 0.0)


def _m6_call(agg, hp, dinv16, b_ml, bn2_g, bn2_b):
    D = NE * HID
    return pl.pallas_call(
        _m6_body,
        grid=(2, NRB),
        in_specs=[
            pl.BlockSpec((2 * NE, RB, HID), lambda p, i: (0, i, 0)),
            pl.BlockSpec((2 * NE, RB, HID), lambda p, i: (0, i, 0)),
            pl.BlockSpec((RB, HID), lambda p, i: (i, 0)),
            pl.BlockSpec((2 * NE, HID), lambda p, i: (0, 0)),
            pl.BlockSpec((1, D), lambda p, i: (0, 0)),
            pl.BlockSpec((1, D), lambda p, i: (0, 0)),
        ],
        out_specs=[
            pl.BlockSpec((RB, D), lambda p, i: (i, 0)),
            pl.BlockSpec((RB, D), lambda p, i: (i, 0)),
        ],
        out_shape=[
            jax.ShapeDtypeStruct((NP, D), _f32),
            jax.ShapeDtypeStruct((NP, D), _f32),
        ],
        scratch_shapes=[pltpu.VMEM((8, D), _f32)],
    )(agg, hp, dinv16, b_ml, bn2_g, bn2_b)


EB = 512
NEB = ET // EB


def _m8_body(r00, r01, r10, r11, b0, w1, b1, w2, b2, out):
    z0 = jnp.maximum(r00[...] + r01[...] + b0[0], 0.0)
    z1 = jnp.maximum(r10[...] + r11[...] + b0[0], 0.0)
    t0 = jnp.maximum(jnp.dot(z0, w1[...], preferred_element_type=_f32) + b1[0], 0.0)
    t1 = jnp.maximum(jnp.dot(z1, w1[...], preferred_element_type=_f32) + b1[0], 0.0)
    out[...] = jnp.dot(0.5 * (t0 + t1), w2[...], preferred_element_type=_f32) + b2[0]


def _m8_call(r00, r01, r10, r11, b_l0, W_l1, b_l1, W_l2p, b_l2p):
    es = pl.BlockSpec((EB, LH), lambda i: (i, 0))
    return pl.pallas_call(
        _m8_body,
        grid=(NEB,),
        in_specs=[
            es, es, es, es,
            pl.BlockSpec((1, LH), lambda i: (0, 0)),
            pl.BlockSpec((LH, LH), lambda i: (0, 0)),
            pl.BlockSpec((1, LH), lambda i: (0, 0)),
            pl.BlockSpec((LH, HID), lambda i: (0, 0)),
            pl.BlockSpec((1, HID), lambda i: (0, 0)),
        ],
        out_specs=pl.BlockSpec((EB, HID), lambda i: (i, 0)),
        out_shape=jax.ShapeDtypeStruct((ET, HID), _f32),
    )(r00, r01, r10, r11, b_l0, W_l1, b_l1, W_l2p, b_l2p)


def _pad_idx(n_extra):
    return (N + (jnp.arange(n_extra, dtype=_i32) % (NP - N))).astype(_i32)


def kernel(x, edge_index, edge_attr, edge_index_test, W_gat, att_src, att_dst,
           b_gat, bn1_g, bn1_b, bn2_g, bn2_b, W_c, b_c, W_mu, b_mu, W_ls, b_ls,
           W_l0, b_l0, W_l1, b_l1, W_l2, b_l2):
    xp = jnp.pad(x, ((0, NP - N), (0, 0)))
    loop = jnp.arange(N, dtype=_i32)
    srcg = jnp.concatenate([edge_index[0], loop, _pad_idx(EG - E - N)])
    dstg = jnp.concatenate([edge_index[1], loop, _pad_idx(EG - E - N)])
    srcw = jnp.concatenate([edge_index[0], _pad_idx(EW - E)])
    dstw = jnp.concatenate([edge_index[1], _pad_idx(EW - E)])
    st = jnp.concatenate([edge_index_test[0], _pad_idx(ET - E)])
    dt = jnp.concatenate([edge_index_test[1], _pad_idx(ET - E)])
    wflat = jnp.pad(edge_attr, ((0, EW - E), (0, L - NE))).reshape(EW * L)
    z128 = jnp.zeros((NP, HID), _f32)

    W_cp = jnp.pad(W_c, ((0, 1), (0, 0), (0, 0)))
    W_mlp = jnp.concatenate([W_mu, W_ls], axis=0)
    b_mlp = jnp.concatenate([b_mu, b_ls], axis=0)
    W0ab = W_l0.reshape(2, NE * HID, LH)
    W_l2p = jnp.pad(W_l2, ((0, 0), (0, HID - 4)))
    b_l2p = jnp.pad(b_l2, (0, HID - 4)).reshape(1, HID)
    b_gat2 = b_gat.reshape(1, -1)
    bn1_g2, bn1_b2 = bn1_g.reshape(1, -1), bn1_b.reshape(1, -1)
    bn2_g2, bn2_b2 = bn2_g.reshape(1, -1), bn2_b.reshape(1, -1)
    b_l0_2, b_l1_2 = b_l0.reshape(1, -1), b_l1.reshape(1, -1)

    xlbm, als128, ald128 = _m1_call(xp, W_gat, att_src, att_dst)
    e_sc = _k1a_call(srcg, dstg, als128, ald128)
    sparts = _k2a_call(dstg, e_sc, z128, EG)
    agg1 = _k3_call(srcg, dstg, e_sc, xlbm.reshape(H * NP, HID), z128,
                    nbp=H, ncol=16, etot=EG)
    h1 = _m2_call(agg1.reshape(H, NP, HID), sparts.reshape(NC, NP, HID),
                  b_gat2, bn1_g2, bn1_b2)

    degp = _k2a_call(dstw, wflat, z128, EW)
    dinv128 = _m2b_call(degp.reshape(NC, NP, HID))
    scale = _k2b_call(srcw, wflat, dinv128)

    hp_c = _mm_call(h1, W_cp, NE + 1, H * HID)
    agg2 = _k3_call(srcw, dstw, scale, hp_c.reshape((NE + 1) * NP, HID), z128,
                    nbp=NE + 1, ncol=16, etot=EW)
    h2 = _m4_call(agg2.reshape(NE + 1, NP, HID), hp_c, dinv128, b_c,
                  bn2_g2, bn2_b2)

    hp_ml = _mm_call(h2, W_mlp, 2 * NE, NE * HID)
    agg3 = _k3_call(srcw, dstw, scale, hp_ml.reshape(2 * NE * NP, HID), z128,
                    nbp=2 * NE, ncol=NE, etot=EW)
    xmu_p, xls_p = _m6_call(agg3.reshape(2 * NE, NP, HID), hp_ml, dinv128,
                            b_mlp, bn2_g2, bn2_b2)

    pq = _mm_call2(xmu_p, W0ab)
    r00, r01, r10, r11 = _k4_call(st, dt, pq.reshape(2 * NP, LH))
    dfull = _m8_call(r00, r01, r10, r11, b_l0_2, W_l1, b_l1_2, W_l2p, b_l2p)

    return (xmu_p[:N], xls_p[:N], dfull[:E, :4])


def _mm2_body(h, w, out):
    out[...] = jnp.dot(h[...], w[0], preferred_element_type=_f32)[None]


def _mm_call2(h, wstack):
    return pl.pallas_call(
        _mm2_body,
        grid=(2, NRB),
        in_specs=[
            pl.BlockSpec((RB, NE * HID), lambda j, r: (r, 0)),
            pl.BlockSpec((1, NE * HID, LH), lambda j, r: (j, 0, 0)),
        ],
        out_specs=pl.BlockSpec((1, RB, LH), lambda j, r: (j, r, 0)),
        out_shape=jax.ShapeDtypeStruct((2, NP, LH), _f32),
    )(h, wstack)


# speedup vs baseline: 6.9040x; 1.0197x over previous
import functools

import jax
import jax.numpy as jnp
from jax import lax
from jax.experimental import pallas as pl
from jax.experimental.pallas import tpu as pltpu
from jax.experimental.pallas import tpu_sc as plsc

N = 10000
NP = 10240
IN = 256
HID = 128
H = 4
NE = 5
LH = 256
E = 160000

NC, NS, L = 2, 16, 16
NW = NC * NS
CH = 128

EG = 172032
EW = 163840
ET = 163840
RPT = NP // NS

_f32 = jnp.float32
_i32 = jnp.int32


def _mesh():
    return plsc.VectorSubcoreMesh(core_axis_name="c", subcore_axis_name="s",
                                  num_cores=NC, num_subcores=NS)


def _k1a_body(src_r, dst_r, als_r, ald_r, e_out, si, di, av, bv, e16, sem):
    cid = lax.axis_index("c")
    sid = lax.axis_index("s")
    w = sid * NC + cid
    nch = EG // (NW * CH)

    def chunk(ch, _):
        base = w * (EG // NW) + ch * CH
        pltpu.sync_copy(src_r.at[pl.ds(base, CH)], si)
        pltpu.sync_copy(dst_r.at[pl.ds(base, CH)], di)
        pltpu.async_copy(als_r.at[si], av, sem).wait()
        pltpu.async_copy(ald_r.at[di], bv, sem).wait()

        def row(r, c):
            v = av[r, pl.ds(0, L)] + bv[r, pl.ds(0, L)]
            v = jnp.where(v >= 0, v, 0.2 * v)
            e16[pl.ds(r * L, L)] = jnp.exp(v)
            return c
        lax.fori_loop(0, CH, row, 0)
        pltpu.sync_copy(e16, e_out.at[pl.ds(base * L, CH * L)])
        return _
    lax.fori_loop(0, nch, chunk, 0)


def _k1a_call(srcg, dstg, als128, ald128):
    f = pl.kernel(
        _k1a_body,
        out_type=jax.ShapeDtypeStruct((EG * L,), _f32),
        mesh=_mesh(),
        scratch_types=[
            pltpu.VMEM((CH,), _i32), pltpu.VMEM((CH,), _i32),
            pltpu.VMEM((CH, HID), _f32), pltpu.VMEM((CH, HID), _f32),
            pltpu.VMEM((CH * L,), _f32),
            pltpu.SemaphoreType.DMA,
        ],
    )
    return f(srcg, dstg, als128, ald128)


def _k2a_body(dst_r, w_r, z_r, d_out, di, wv, wv128, dacc, *, etot):
    cid = lax.axis_index("c")
    sid = lax.axis_index("s")
    w = sid * NC + cid
    rows0 = sid * RPT
    pltpu.sync_copy(z_r.at[pl.ds(rows0, RPT)], dacc.at[pl.ds(rows0, RPT)])

    def zrow(r, c):
        for j in range(1, HID // L):
            wv128[r, pl.ds(j * L, L)] = jnp.zeros((L,), _f32)
        return c
    lax.fori_loop(0, CH, zrow, 0)
    plsc.subcore_barrier()
    nch = etot // (NW * CH)

    def chunk(ch, _):
        base = w * (etot // NW) + ch * CH
        pltpu.sync_copy(dst_r.at[pl.ds(base, CH)], di)
        pltpu.sync_copy(w_r.at[pl.ds(base * L, CH * L)], wv)

        def row(r, c):
            wv128[r, pl.ds(0, L)] = wv[pl.ds(r * L, L)]
            return c
        lax.fori_loop(0, CH, row, 0)
        pltpu.sync_copy(wv128, dacc.at[di], add=True)
        return _
    lax.fori_loop(0, nch, chunk, 0)
    plsc.subcore_barrier()
    pltpu.sync_copy(dacc.at[pl.ds(rows0, RPT)],
                    d_out.at[pl.ds(cid * NP + rows0, RPT)])


def _k2a_call(dst, wflat, z128, etot):
    body = functools.partial(_k2a_body, etot=etot)
    f = pl.kernel(
        body,
        out_type=jax.ShapeDtypeStruct((NC * NP, HID), _f32),
        mesh=_mesh(),
        scratch_types=[
            pltpu.VMEM((CH,), _i32), pltpu.VMEM((CH * L,), _f32),
            pltpu.VMEM((CH, HID), _f32),
            pltpu.VMEM_SHARED((NP, HID), _f32),
        ],
    )
    return f(dst, wflat, z128)


def _k2b_body(src_r, w_r, dinv_r, s_out, si, wv, dv, sem):
    cid = lax.axis_index("c")
    sid = lax.axis_index("s")
    w = sid * NC + cid
    nch = EW // (NW * CH)

    def chunk(ch, _):
        base = w * (EW // NW) + ch * CH
        pltpu.sync_copy(src_r.at[pl.ds(base, CH)], si)
        pltpu.sync_copy(w_r.at[pl.ds(base * L, CH * L)], wv)
        pltpu.async_copy(dinv_r.at[si], dv, sem).wait()

        def row(r, c):
            wv[pl.ds(r * L, L)] = wv[pl.ds(r * L, L)] * dv[r, pl.ds(0, L)]
            return c
        lax.fori_loop(0, CH, row, 0)
        pltpu.sync_copy(wv, s_out.at[pl.ds(base * L, CH * L)])
        return _
    lax.fori_loop(0, nch, chunk, 0)


def _k2b_call(srcw, wflat, dinv128):
    f = pl.kernel(
        _k2b_body,
        out_type=jax.ShapeDtypeStruct((EW * L,), _f32),
        mesh=_mesh(),
        scratch_types=[
            pltpu.VMEM((CH,), _i32), pltpu.VMEM((CH * L,), _f32),
            pltpu.VMEM((CH, HID), _f32), pltpu.SemaphoreType.DMA,
        ],
    )
    return f(srcw, wflat, dinv128)


def _k3_body(src_r, dst_r, sc_r, tab_r, z_r, out_r,
             si, di, gi, sv, rows, acc, sem, *, nbp, ncol, etot):
    cid = lax.axis_index("c")
    sid = lax.axis_index("s")
    rows0 = sid * RPT
    ept = etot // NS
    nch = ept // CH
    def block(k, carry):
        b = k * NC + cid
        boff = b * NP
        colv = jnp.full((L,), lax.rem(b, jnp.int32(ncol)), _i32)
        pltpu.sync_copy(z_r.at[pl.ds(rows0, RPT)], acc.at[pl.ds(rows0, RPT)])
        plsc.subcore_barrier()

        def chunk(ch, _):
            base = sid * ept + ch * CH
            pltpu.sync_copy(src_r.at[pl.ds(base, CH)], si)
            pltpu.sync_copy(dst_r.at[pl.ds(base, CH)], di)
            pltpu.sync_copy(sc_r.at[pl.ds(base * L, CH * L)], sv)

            def addi(r, c):
                gi[pl.ds(r * L, L)] = si[pl.ds(r * L, L)] + boff
                return c
            lax.fori_loop(0, CH // L, addi, 0)
            pltpu.async_copy(tab_r.at[gi], rows, sem).wait()

            def edge(e4, c):
                for u in range(4):
                    e = e4 * 4 + u
                    v = sv[pl.ds(e * L, L)]
                    sp = v[colv]
                    for j in range(HID // L):
                        rows[e, pl.ds(j * L, L)] = rows[e, pl.ds(j * L, L)] * sp
                return c
            lax.fori_loop(0, CH // 4, edge, 0)
            pltpu.sync_copy(rows, acc.at[di], add=True)
            return _
        lax.fori_loop(0, nch, chunk, 0)
        plsc.subcore_barrier()
        pltpu.sync_copy(acc.at[pl.ds(rows0, RPT)],
                        out_r.at[pl.ds(boff + rows0, RPT)])
        plsc.subcore_barrier()
        return carry
    lax.fori_loop(0, (nbp - cid + NC - 1) // NC, block, 0)


def _k3_call(src, dst, scale, table, z128, *, nbp, ncol, etot):
    body = functools.partial(_k3_body, nbp=nbp, ncol=ncol, etot=etot)
    f = pl.kernel(
        body,
        out_type=jax.ShapeDtypeStruct((nbp * NP, HID), _f32),
        mesh=_mesh(),
        scratch_types=[
            pltpu.VMEM((CH,), _i32), pltpu.VMEM((CH,), _i32),
            pltpu.VMEM((CH,), _i32), pltpu.VMEM((CH * L,), _f32),
            pltpu.VMEM((CH, HID), _f32),
            pltpu.VMEM_SHARED((NP, HID), _f32),
            pltpu.SemaphoreType.DMA,
        ],
    )
    return f(src, dst, scale, table, z128)


CH4 = 64


def _k4_body(st_r, dt_r, pq_r, r00, r01, r10, r11,
             si, di, gis, gid, av, bv, cv, dv, semg, semw):
    cid = lax.axis_index("c")
    sid = lax.axis_index("s")
    w = sid * NC + cid
    nch = ET // (NW * CH4)

    def chunk(ch, _):
        base = w * (ET // NW) + ch * CH4
        pltpu.sync_copy(st_r.at[pl.ds(base, CH4)], si)
        pltpu.sync_copy(dt_r.at[pl.ds(base, CH4)], di)

        def addi(r, c):
            gis[pl.ds(r * L, L)] = si[pl.ds(r * L, L)] + NP
            gid[pl.ds(r * L, L)] = di[pl.ds(r * L, L)] + NP
            return c
        lax.fori_loop(0, CH4 // L, addi, 0)
        c0 = pltpu.async_copy(pq_r.at[si], av, semg)
        c1 = pltpu.async_copy(pq_r.at[gid], bv, semg)
        c2 = pltpu.async_copy(pq_r.at[di], cv, semg)
        c3 = pltpu.async_copy(pq_r.at[gis], dv, semg)
        c0.wait(); c1.wait(); c2.wait(); c3.wait()
        w0 = pltpu.async_copy(av, r00.at[pl.ds(base, CH4)], semw)
        w1 = pltpu.async_copy(bv, r01.at[pl.ds(base, CH4)], semw)
        w2 = pltpu.async_copy(cv, r10.at[pl.ds(base, CH4)], semw)
        w3 = pltpu.async_copy(dv, r11.at[pl.ds(base, CH4)], semw)
        w0.wait(); w1.wait(); w2.wait(); w3.wait()
        return _
    lax.fori_loop(0, nch, chunk, 0)


def _k4_call(st, dt, pq):
    o = jax.ShapeDtypeStruct((ET, LH), _f32)
    f = pl.kernel(
        _k4_body,
        out_type=(o, o, o, o),
        mesh=_mesh(),
        scratch_types=[
            pltpu.VMEM((CH4,), _i32), pltpu.VMEM((CH4,), _i32),
            pltpu.VMEM((CH4,), _i32), pltpu.VMEM((CH4,), _i32),
            pltpu.VMEM((CH4, LH), _f32), pltpu.VMEM((CH4, LH), _f32),
            pltpu.VMEM((CH4, LH), _f32), pltpu.VMEM((CH4, LH), _f32),
            pltpu.SemaphoreType.DMA, pltpu.SemaphoreType.DMA,
        ],
    )
    return f(st, dt, pq)


RB = 256
NRB = NP // RB


def _m1_body(xp, wg, ats, atd, xlbm, als128, ald128):
    a = jnp.dot(xp[...], wg[...], preferred_element_type=_f32)
    a3 = a.reshape(RB, H, HID)
    xlbm[...] = a3.swapaxes(0, 1)
    als = (a3 * ats[...][None]).sum(-1)
    ald = (a3 * atd[...][None]).sum(-1)
    z = jnp.zeros((RB, HID - H), _f32)
    als128[...] = jnp.concatenate([als, z], axis=1)
    ald128[...] = jnp.concatenate([ald, z], axis=1)


def _m1_call(xp, W_gat, att_src, att_dst):
    return pl.pallas_call(
        _m1_body,
        grid=(NRB,),
        in_specs=[
            pl.BlockSpec((RB, IN), lambda i: (i, 0)),
            pl.BlockSpec((IN, H * HID), lambda i: (0, 0)),
            pl.BlockSpec((H, HID), lambda i: (0, 0)),
            pl.BlockSpec((H, HID), lambda i: (0, 0)),
        ],
        out_specs=[
            pl.BlockSpec((H, RB, HID), lambda i: (0, i, 0)),
            pl.BlockSpec((RB, HID), lambda i: (i, 0)),
            pl.BlockSpec((RB, HID), lambda i: (i, 0)),
        ],
        out_shape=[
            jax.ShapeDtypeStruct((H, NP, HID), _f32),
            jax.ShapeDtypeStruct((NP, HID), _f32),
            jax.ShapeDtypeStruct((NP, HID), _f32),
        ],
    )(xp, W_gat, att_src, att_dst)


def _m2b_body(degp, dinv128):
    d = degp[0] + degp[1] + 1.0
    dinv128[...] = lax.rsqrt(d)


def _m2b_call(degp):
    return pl.pallas_call(
        _m2b_body,
        grid=(NRB,),
        in_specs=[pl.BlockSpec((NC, RB, HID), lambda i: (0, i, 0))],
        out_specs=pl.BlockSpec((RB, HID), lambda i: (i, 0)),
        out_shape=jax.ShapeDtypeStruct((NP, HID), _f32),
    )(degp)


def _rowmask(i):
    rows = i * RB + lax.broadcasted_iota(_i32, (RB, 1), 0)
    return rows < N


def _m2_body(agg, sp, bg, g, b, out, acc):
    p = pl.program_id(0)
    i = pl.program_id(1)
    s = sp[0] + sp[1]
    s4 = s[:, :H]
    at = agg[...].swapaxes(0, 1)
    y3 = at / (s4[:, :, None] + 1e-16)
    y = y3.reshape(RB, H * HID) + bg[0]

    @pl.when(p == 0)
    def _():
        @pl.when(i == 0)
        def _():
            acc[...] = jnp.zeros_like(acc)
        acc[0] = acc[0] + y.sum(0)
        acc[1] = acc[1] + (y * y).sum(0)

    @pl.when(p == 1)
    def _():
        mean = acc[0] / N
        var = acc[1] / N - mean * mean
        yn = (y - mean) * lax.rsqrt(var + 1e-5) * g[0] + b[0]
        out[...] = jnp.where(_rowmask(i), jnp.maximum(yn, 0.0), 0.0)


def _m2_call(agg, sparts, b_gat, bn1_g, bn1_b):
    D = H * HID
    return pl.pallas_call(
        _m2_body,
        grid=(2, NRB),
        in_specs=[
            pl.BlockSpec((H, RB, HID), lambda p, i: (0, i, 0)),
            pl.BlockSpec((NC, RB, HID), lambda p, i: (0, i, 0)),
            pl.BlockSpec((1, D), lambda p, i: (0, 0)),
            pl.BlockSpec((1, D), lambda p, i: (0, 0)),
            pl.BlockSpec((1, D), lambda p, i: (0, 0)),
        ],
        out_specs=pl.BlockSpec((RB, D), lambda p, i: (i, 0)),
        out_shape=jax.ShapeDtypeStruct((NP, D), _f32),
        scratch_shapes=[pltpu.VMEM((8, D), _f32)],
    )(agg, sparts, b_gat, bn1_g, bn1_b)


def _mm_body(h, w, out):
    out[...] = jnp.dot(h[...], w[0], preferred_element_type=_f32)[None]


def _mm_call(h, wstack, nb, din):
    return pl.pallas_call(
        _mm_body,
        grid=(nb, NRB),
        in_specs=[
            pl.BlockSpec((RB, din), lambda j, r: (r, 0)),
            pl.BlockSpec((1, din, HID), lambda j, r: (j, 0, 0)),
        ],
        out_specs=pl.BlockSpec((1, RB, HID), lambda j, r: (j, r, 0)),
        out_shape=jax.ShapeDtypeStruct((nb, NP, HID), _f32),
    )(h, wstack)


def _m4_body(agg, hp, dinv, bc, g, b, out, acc):
    p = pl.program_id(0)
    i = pl.program_id(1)
    dv = dinv[:, :NE]
    at = agg[...].swapaxes(0, 1)
    ht = hp[...].swapaxes(0, 1)
    y3 = at * dv[:, :, None] + ht * (dv * dv)[:, :, None] + bc[...][None]
    y = y3.reshape(RB, NE * HID)

    @pl.when(p == 0)
    def _():
        @pl.when(i == 0)
        def _():
            acc[...] = jnp.zeros_like(acc)
        acc[0] = acc[0] + y.sum(0)
        acc[1] = acc[1] + (y * y).sum(0)

    @pl.when(p == 1)
    def _():
        mean = acc[0] / N
        var = acc[1] / N - mean * mean
        yn = (y - mean) * lax.rsqrt(var + 1e-5) * g[0] + b[0]
        out[...] = jnp.where(_rowmask(i), jnp.maximum(yn, 0.0), 0.0)


def _m4_call(agg, hp, dinv16, b_c, bn2_g, bn2_b):
    D = NE * HID
    return pl.pallas_call(
        _m4_body,
        grid=(2, NRB),
        in_specs=[
            pl.BlockSpec((NE, RB, HID), lambda p, i: (0, i, 0)),
            pl.BlockSpec((NE, RB, HID), lambda p, i: (0, i, 0)),
            pl.BlockSpec((RB, HID), lambda p, i: (i, 0)),
            pl.BlockSpec((NE, HID), lambda p, i: (0, 0)),
            pl.BlockSpec((1, D), lambda p, i: (0, 0)),
            pl.BlockSpec((1, D), lambda p, i: (0, 0)),
        ],
        out_specs=pl.BlockSpec((RB, D), lambda p, i: (i, 0)),
        out_shape=jax.ShapeDtypeStruct((NP, D), _f32),
        scratch_shapes=[pltpu.VMEM((8, D), _f32)],
    )(agg, hp, dinv16, b_c, bn2_g, bn2_b)


def _m6_body(agg, hp, dinv, bml, g, b, omu, ols, acc):
    p = pl.program_id(0)
    i = pl.program_id(1)
    dv = dinv[:, :NE]
    dvv = jnp.concatenate([dv, dv], axis=1)
    at = agg[...].swapaxes(0, 1)
    ht = hp[...].swapaxes(0, 1)
    y3 = at * dvv[:, :, None] + ht * (dvv * dvv)[:, :, None] + bml[...][None]
    D = NE * HID
    ymu = y3[:, :NE].reshape(RB, D)
    yls = y3[:, NE:].reshape(RB, D)

    @pl.when(p == 0)
    def _():
        @pl.when(i == 0)
        def _():
            acc[...] = jnp.zeros_like(acc)
        acc[0] = acc[0] + ymu.sum(0)
        acc[1] = acc[1] + (ymu * ymu).sum(0)
        acc[2] = acc[2] + yls.sum(0)
        acc[3] = acc[3] + (yls * yls).sum(0)

    @pl.when(p == 1)
    def _():
        m = _rowmask(i)
        mu_m = acc[0] / N
        mu_v = acc[1] / N - mu_m * mu_m
        ls_m = acc[2] / N
        ls_v = acc[3] / N - ls_m * ls_m
        a = (ymu - mu_m) * lax.rsqrt(mu_v + 1e-5) * g[0] + b[0]
        c = (yls - ls_m) * lax.rsqrt(ls_v + 1e-5) * g[0] + b[0]
        omu[...] = jnp.where(m, jnp.maximum(a, 0.0), 0.0)
        ols[...] = jnp.where(m, jnp.maximum(c, 0.0), 0.0)


def _m6_call(agg, hp, dinv16, b_ml, bn2_g, bn2_b):
    D = NE * HID
    return pl.pallas_call(
        _m6_body,
        grid=(2, NRB),
        in_specs=[
            pl.BlockSpec((2 * NE, RB, HID), lambda p, i: (0, i, 0)),
            pl.BlockSpec((2 * NE, RB, HID), lambda p, i: (0, i, 0)),
            pl.BlockSpec((RB, HID), lambda p, i: (i, 0)),
            pl.BlockSpec((2 * NE, HID), lambda p, i: (0, 0)),
            pl.BlockSpec((1, D), lambda p, i: (0, 0)),
            pl.BlockSpec((1, D), lambda p, i: (0, 0)),
        ],
        out_specs=[
            pl.BlockSpec((RB, D), lambda p, i: (i, 0)),
            pl.BlockSpec((RB, D), lambda p, i: (i, 0)),
        ],
        out_shape=[
            jax.ShapeDtypeStruct((NP, D), _f32),
            jax.ShapeDtypeStruct((NP, D), _f32),
        ],
        scratch_shapes=[pltpu.VMEM((8, D), _f32)],
    )(agg, hp, dinv16, b_ml, bn2_g, bn2_b)


EB = 512
NEB = ET // EB


def _m8_body(r00, r01, r10, r11, b0, w1, b1, w2, b2, out):
    z0 = jnp.maximum(r00[...] + r01[...] + b0[0], 0.0)
    z1 = jnp.maximum(r10[...] + r11[...] + b0[0], 0.0)
    t0 = jnp.maximum(jnp.dot(z0, w1[...], preferred_element_type=_f32) + b1[0], 0.0)
    t1 = jnp.maximum(jnp.dot(z1, w1[...], preferred_element_type=_f32) + b1[0], 0.0)
    out[...] = jnp.dot(0.5 * (t0 + t1), w2[...], preferred_element_type=_f32) + b2[0]


def _m8_call(r00, r01, r10, r11, b_l0, W_l1, b_l1, W_l2p, b_l2p):
    es = pl.BlockSpec((EB, LH), lambda i: (i, 0))
    return pl.pallas_call(
        _m8_body,
        grid=(NEB,),
        in_specs=[
            es, es, es, es,
            pl.BlockSpec((1, LH), lambda i: (0, 0)),
            pl.BlockSpec((LH, LH), lambda i: (0, 0)),
            pl.BlockSpec((1, LH), lambda i: (0, 0)),
            pl.BlockSpec((LH, HID), lambda i: (0, 0)),
            pl.BlockSpec((1, HID), lambda i: (0, 0)),
        ],
        out_specs=pl.BlockSpec((EB, HID), lambda i: (i, 0)),
        out_shape=jax.ShapeDtypeStruct((ET, HID), _f32),
    )(r00, r01, r10, r11, b_l0, W_l1, b_l1, W_l2p, b_l2p)


def _pad_idx(n_extra):
    return (N + (jnp.arange(n_extra, dtype=_i32) % (NP - N))).astype(_i32)


def kernel(x, edge_index, edge_attr, edge_index_test, W_gat, att_src, att_dst,
           b_gat, bn1_g, bn1_b, bn2_g, bn2_b, W_c, b_c, W_mu, b_mu, W_ls, b_ls,
           W_l0, b_l0, W_l1, b_l1, W_l2, b_l2):
    xp = jnp.pad(x, ((0, NP - N), (0, 0)))
    loop = jnp.arange(N, dtype=_i32)
    srcg = jnp.concatenate([edge_index[0], loop, _pad_idx(EG - E - N)])
    dstg = jnp.concatenate([edge_index[1], loop, _pad_idx(EG - E - N)])
    srcw = jnp.concatenate([edge_index[0], _pad_idx(EW - E)])
    dstw = jnp.concatenate([edge_index[1], _pad_idx(EW - E)])
    st = jnp.concatenate([edge_index_test[0], _pad_idx(ET - E)])
    dt = jnp.concatenate([edge_index_test[1], _pad_idx(ET - E)])
    wflat = jnp.pad(edge_attr, ((0, EW - E), (0, L - NE))).reshape(EW * L)
    z128 = jnp.zeros((NP, HID), _f32)

    W_mlp = jnp.concatenate([W_mu, W_ls], axis=0)
    b_mlp = jnp.concatenate([b_mu, b_ls], axis=0)
    W0ab = W_l0.reshape(2, NE * HID, LH)
    W_l2p = jnp.pad(W_l2, ((0, 0), (0, HID - 4)))
    b_l2p = jnp.pad(b_l2, (0, HID - 4)).reshape(1, HID)
    b_gat2 = b_gat.reshape(1, -1)
    bn1_g2, bn1_b2 = bn1_g.reshape(1, -1), bn1_b.reshape(1, -1)
    bn2_g2, bn2_b2 = bn2_g.reshape(1, -1), bn2_b.reshape(1, -1)
    b_l0_2, b_l1_2 = b_l0.reshape(1, -1), b_l1.reshape(1, -1)

    xlbm, als128, ald128 = _m1_call(xp, W_gat, att_src, att_dst)
    e_sc = _k1a_call(srcg, dstg, als128, ald128)
    sparts = _k2a_call(dstg, e_sc, z128, EG)
    agg1 = _k3_call(srcg, dstg, e_sc, xlbm.reshape(H * NP, HID), z128,
                    nbp=H, ncol=16, etot=EG)
    h1 = _m2_call(agg1.reshape(H, NP, HID), sparts.reshape(NC, NP, HID),
                  b_gat2, bn1_g2, bn1_b2)

    degp = _k2a_call(dstw, wflat, z128, EW)
    dinv128 = _m2b_call(degp.reshape(NC, NP, HID))
    scale = _k2b_call(srcw, wflat, dinv128)

    hp_c = _mm_call(h1, W_c, NE, H * HID)
    agg2 = _k3_call(srcw, dstw, scale, hp_c.reshape(NE * NP, HID), z128,
                    nbp=NE, ncol=16, etot=EW)
    h2 = _m4_call(agg2.reshape(NE, NP, HID), hp_c, dinv128, b_c,
                  bn2_g2, bn2_b2)

    hp_ml = _mm_call(h2, W_mlp, 2 * NE, NE * HID)
    agg3 = _k3_call(srcw, dstw, scale, hp_ml.reshape(2 * NE * NP, HID), z128,
                    nbp=2 * NE, ncol=NE, etot=EW)
    xmu_p, xls_p = _m6_call(agg3.reshape(2 * NE, NP, HID), hp_ml, dinv128,
                            b_mlp, bn2_g2, bn2_b2)

    pq = _mm_call2(xmu_p, W0ab)
    r00, r01, r10, r11 = _k4_call(st, dt, pq.reshape(2 * NP, LH))
    dfull = _m8_call(r00, r01, r10, r11, b_l0_2, W_l1, b_l1_2, W_l2p, b_l2p)

    return (xmu_p[:N], xls_p[:N], dfull[:E, :4])


def _mm2_body(h, w, out):
    out[...] = jnp.dot(h[...], w[0], preferred_element_type=_f32)[None]


def _mm_call2(h, wstack):
    return pl.pallas_call(
        _mm2_body,
        grid=(2, NRB),
        in_specs=[
            pl.BlockSpec((RB, NE * HID), lambda j, r: (r, 0)),
            pl.BlockSpec((1, NE * HID, LH), lambda j, r: (j, 0, 0)),
        ],
        out_specs=pl.BlockSpec((1, RB, LH), lambda j, r: (j, r, 0)),
        out_shape=jax.ShapeDtypeStruct((2, NP, LH), _f32),
    )(h, wstack)


# speedup vs baseline: 8.2335x; 1.1926x over previous
import functools

import jax
import jax.numpy as jnp
from jax import lax
from jax.experimental import pallas as pl
from jax.experimental.pallas import tpu as pltpu
from jax.experimental.pallas import tpu_sc as plsc

N = 10000
NP = 10240
IN = 256
HID = 128
H = 4
NE = 5
LH = 256
E = 160000

NC, NS, L = 2, 16, 16
NW = NC * NS
CH = 128

EG = 172032
EW = 163840
ET = 163840
RPT = NP // NS

_f32 = jnp.float32
_i32 = jnp.int32


def _mesh():
    return plsc.VectorSubcoreMesh(core_axis_name="c", subcore_axis_name="s",
                                  num_cores=NC, num_subcores=NS)


def _k1a_body(src_r, dst_r, als_r, ald_r, e_out, si, di, av, bv, e16, sem):
    cid = lax.axis_index("c")
    sid = lax.axis_index("s")
    w = sid * NC + cid
    nch = EG // (NW * CH)

    def chunk(ch, _):
        base = w * (EG // NW) + ch * CH
        pltpu.sync_copy(src_r.at[pl.ds(base, CH)], si)
        pltpu.sync_copy(dst_r.at[pl.ds(base, CH)], di)
        c0 = pltpu.async_copy(als_r.at[si], av, sem)
        c1 = pltpu.async_copy(ald_r.at[di], bv, sem)
        c0.wait()
        c1.wait()

        def row(r, c):
            v = av[r, pl.ds(0, L)] + bv[r, pl.ds(0, L)]
            v = jnp.where(v >= 0, v, 0.2 * v)
            e16[pl.ds(r * L, L)] = jnp.exp(v)
            return c
        lax.fori_loop(0, CH, row, 0)
        pltpu.sync_copy(e16, e_out.at[pl.ds(base * L, CH * L)])
        return _
    lax.fori_loop(0, nch, chunk, 0)


def _k1a_call(srcg, dstg, als128, ald128):
    f = pl.kernel(
        _k1a_body,
        out_type=jax.ShapeDtypeStruct((EG * L,), _f32),
        mesh=_mesh(),
        scratch_types=[
            pltpu.VMEM((CH,), _i32), pltpu.VMEM((CH,), _i32),
            pltpu.VMEM((CH, HID), _f32), pltpu.VMEM((CH, HID), _f32),
            pltpu.VMEM((CH * L,), _f32),
            pltpu.SemaphoreType.DMA,
        ],
    )
    return f(srcg, dstg, als128, ald128)


def _k2a_body(dst_r, w_r, z_r, d_out, di, wv, wv128, dacc, *, etot):
    cid = lax.axis_index("c")
    sid = lax.axis_index("s")
    w = sid * NC + cid
    rows0 = sid * RPT
    pltpu.sync_copy(z_r.at[pl.ds(rows0, RPT)], dacc.at[pl.ds(rows0, RPT)])

    def zrow(r, c):
        for j in range(1, HID // L):
            wv128[r, pl.ds(j * L, L)] = jnp.zeros((L,), _f32)
        return c
    lax.fori_loop(0, CH, zrow, 0)
    plsc.subcore_barrier()
    nch = etot // (NW * CH)

    def chunk(ch, _):
        base = w * (etot // NW) + ch * CH
        pltpu.sync_copy(dst_r.at[pl.ds(base, CH)], di)
        pltpu.sync_copy(w_r.at[pl.ds(base * L, CH * L)], wv)

        def row(r, c):
            wv128[r, pl.ds(0, L)] = wv[pl.ds(r * L, L)]
            return c
        lax.fori_loop(0, CH, row, 0)
        pltpu.sync_copy(wv128, dacc.at[di], add=True)
        return _
    lax.fori_loop(0, nch, chunk, 0)
    plsc.subcore_barrier()
    pltpu.sync_copy(dacc.at[pl.ds(rows0, RPT)],
                    d_out.at[pl.ds(cid * NP + rows0, RPT)])


def _k2a_call(dst, wflat, z128, etot):
    body = functools.partial(_k2a_body, etot=etot)
    f = pl.kernel(
        body,
        out_type=jax.ShapeDtypeStruct((NC * NP, HID), _f32),
        mesh=_mesh(),
        scratch_types=[
            pltpu.VMEM((CH,), _i32), pltpu.VMEM((CH * L,), _f32),
            pltpu.VMEM((CH, HID), _f32),
            pltpu.VMEM_SHARED((NP, HID), _f32),
        ],
    )
    return f(dst, wflat, z128)


def _k2b_body(src_r, w_r, dinv_r, s_out, si, wv, dv, sem):
    cid = lax.axis_index("c")
    sid = lax.axis_index("s")
    w = sid * NC + cid
    nch = EW // (NW * CH)

    def chunk(ch, _):
        base = w * (EW // NW) + ch * CH
        pltpu.sync_copy(src_r.at[pl.ds(base, CH)], si)
        pltpu.sync_copy(w_r.at[pl.ds(base * L, CH * L)], wv)
        pltpu.async_copy(dinv_r.at[si], dv, sem).wait()

        def row(r, c):
            wv[pl.ds(r * L, L)] = wv[pl.ds(r * L, L)] * dv[r, pl.ds(0, L)]
            return c
        lax.fori_loop(0, CH, row, 0)
        pltpu.sync_copy(wv, s_out.at[pl.ds(base * L, CH * L)])
        return _
    lax.fori_loop(0, nch, chunk, 0)


def _k2b_call(srcw, wflat, dinv128):
    f = pl.kernel(
        _k2b_body,
        out_type=jax.ShapeDtypeStruct((EW * L,), _f32),
        mesh=_mesh(),
        scratch_types=[
            pltpu.VMEM((CH,), _i32), pltpu.VMEM((CH * L,), _f32),
            pltpu.VMEM((CH, HID), _f32), pltpu.SemaphoreType.DMA,
        ],
    )
    return f(srcw, wflat, dinv128)


def _k3_body(src_r, dst_r, sc_r, tab_r, z_r, out_r,
             si0, si1, di0, di1, gi0, gi1, sv0, sv1, rw0, rw1, acc,
             sem0, sem1, *, nbp, ncol, etot):
    cid = lax.axis_index("c")
    sid = lax.axis_index("s")
    rows0 = sid * RPT
    ept = etot // NS
    nch = ept // CH
    sis, dis, gis, svs, rws = [si0, si1], [di0, di1], [gi0, gi1], \
        [sv0, sv1], [rw0, rw1]
    sems = [sem0, sem1]

    def stage(c, b, boff):
        base = sid * ept + c * CH
        pltpu.sync_copy(src_r.at[pl.ds(base, CH)], sis[b])
        pltpu.sync_copy(dst_r.at[pl.ds(base, CH)], dis[b])
        pltpu.sync_copy(sc_r.at[pl.ds(base * L, CH * L)], svs[b])

        def addi(r, cc):
            gis[b][pl.ds(r * L, L)] = sis[b][pl.ds(r * L, L)] + boff
            return cc
        lax.fori_loop(0, CH // L, addi, 0)
        pltpu.async_copy(tab_r.at[gis[b]], rws[b], sems[b])

    def block(k, carry):
        b = k * NC + cid
        boff = b * NP
        colv = jnp.full((L,), lax.rem(b, jnp.int32(ncol)), _i32)
        pltpu.sync_copy(z_r.at[pl.ds(rows0, RPT)], acc.at[pl.ds(rows0, RPT)])
        plsc.subcore_barrier()
        stage(0, 0, boff)

        def chunk2(cc, carry2):
            for bb in range(2):
                c = cc * 2 + bb

                @pl.when(c + 1 < nch)
                def _stage_next():
                    stage(c + 1, 1 - bb, boff)
                pltpu.make_async_copy(tab_r.at[gis[bb]], rws[bb],
                                      sems[bb]).wait()

                def edge(e4, c2):
                    for u in range(4):
                        e = e4 * 4 + u
                        v = svs[bb][pl.ds(e * L, L)]
                        sp = v[colv]
                        for j in range(HID // L):
                            rws[bb][e, pl.ds(j * L, L)] = \
                                rws[bb][e, pl.ds(j * L, L)] * sp
                    return c2
                lax.fori_loop(0, CH // 4, edge, 0)
                pltpu.sync_copy(rws[bb], acc.at[dis[bb]], add=True)
            return carry2
        lax.fori_loop(0, nch // 2, chunk2, 0)
        plsc.subcore_barrier()
        pltpu.sync_copy(acc.at[pl.ds(rows0, RPT)],
                        out_r.at[pl.ds(boff + rows0, RPT)])
        plsc.subcore_barrier()
        return carry
    lax.fori_loop(0, (nbp - cid + NC - 1) // NC, block, 0)


def _k3_call(src, dst, scale, table, z128, *, nbp, ncol, etot):
    body = functools.partial(_k3_body, nbp=nbp, ncol=ncol, etot=etot)
    f = pl.kernel(
        body,
        out_type=jax.ShapeDtypeStruct((nbp * NP, HID), _f32),
        mesh=_mesh(),
        scratch_types=[
            pltpu.VMEM((CH,), _i32), pltpu.VMEM((CH,), _i32),
            pltpu.VMEM((CH,), _i32), pltpu.VMEM((CH,), _i32),
            pltpu.VMEM((CH,), _i32), pltpu.VMEM((CH,), _i32),
            pltpu.VMEM((CH * L,), _f32), pltpu.VMEM((CH * L,), _f32),
            pltpu.VMEM((CH, HID), _f32), pltpu.VMEM((CH, HID), _f32),
            pltpu.VMEM_SHARED((NP, HID), _f32),
            pltpu.SemaphoreType.DMA, pltpu.SemaphoreType.DMA,
        ],
    )
    return f(src, dst, scale, table, z128)


CH4 = 64


def _k4_body(st_r, dt_r, pq_r, r00, r01, r10, r11,
             si, di, gis, gid, av, bv, cv, dv, semg, semw):
    cid = lax.axis_index("c")
    sid = lax.axis_index("s")
    w = sid * NC + cid
    nch = ET // (NW * CH4)

    def chunk(ch, _):
        base = w * (ET // NW) + ch * CH4
        pltpu.sync_copy(st_r.at[pl.ds(base, CH4)], si)
        pltpu.sync_copy(dt_r.at[pl.ds(base, CH4)], di)

        def addi(r, c):
            gis[pl.ds(r * L, L)] = si[pl.ds(r * L, L)] + NP
            gid[pl.ds(r * L, L)] = di[pl.ds(r * L, L)] + NP
            return c
        lax.fori_loop(0, CH4 // L, addi, 0)
        c0 = pltpu.async_copy(pq_r.at[si], av, semg)
        c1 = pltpu.async_copy(pq_r.at[gid], bv, semg)
        c2 = pltpu.async_copy(pq_r.at[di], cv, semg)
        c3 = pltpu.async_copy(pq_r.at[gis], dv, semg)
        c0.wait(); c1.wait(); c2.wait(); c3.wait()
        w0 = pltpu.async_copy(av, r00.at[pl.ds(base, CH4)], semw)
        w1 = pltpu.async_copy(bv, r01.at[pl.ds(base, CH4)], semw)
        w2 = pltpu.async_copy(cv, r10.at[pl.ds(base, CH4)], semw)
        w3 = pltpu.async_copy(dv, r11.at[pl.ds(base, CH4)], semw)
        w0.wait(); w1.wait(); w2.wait(); w3.wait()
        return _
    lax.fori_loop(0, nch, chunk, 0)


def _k4_call(st, dt, pq):
    o = jax.ShapeDtypeStruct((ET, LH), _f32)
    f = pl.kernel(
        _k4_body,
        out_type=(o, o, o, o),
        mesh=_mesh(),
        scratch_types=[
            pltpu.VMEM((CH4,), _i32), pltpu.VMEM((CH4,), _i32),
            pltpu.VMEM((CH4,), _i32), pltpu.VMEM((CH4,), _i32),
            pltpu.VMEM((CH4, LH), _f32), pltpu.VMEM((CH4, LH), _f32),
            pltpu.VMEM((CH4, LH), _f32), pltpu.VMEM((CH4, LH), _f32),
            pltpu.SemaphoreType.DMA, pltpu.SemaphoreType.DMA,
        ],
    )
    return f(st, dt, pq)


RB = 256
NRB = NP // RB


def _m1_body(xp, wg, ats, atd, xlbm, als128, ald128):
    a = jnp.dot(xp[...], wg[...], preferred_element_type=_f32)
    a3 = a.reshape(RB, H, HID)
    xlbm[...] = a3.swapaxes(0, 1)
    als = (a3 * ats[...][None]).sum(-1)
    ald = (a3 * atd[...][None]).sum(-1)
    z = jnp.zeros((RB, HID - H), _f32)
    als128[...] = jnp.concatenate([als, z], axis=1)
    ald128[...] = jnp.concatenate([ald, z], axis=1)


def _m1_call(xp, W_gat, att_src, att_dst):
    return pl.pallas_call(
        _m1_body,
        grid=(NRB,),
        in_specs=[
            pl.BlockSpec((RB, IN), lambda i: (i, 0)),
            pl.BlockSpec((IN, H * HID), lambda i: (0, 0)),
            pl.BlockSpec((H, HID), lambda i: (0, 0)),
            pl.BlockSpec((H, HID), lambda i: (0, 0)),
        ],
        out_specs=[
            pl.BlockSpec((H, RB, HID), lambda i: (0, i, 0)),
            pl.BlockSpec((RB, HID), lambda i: (i, 0)),
            pl.BlockSpec((RB, HID), lambda i: (i, 0)),
        ],
        out_shape=[
            jax.ShapeDtypeStruct((H, NP, HID), _f32),
            jax.ShapeDtypeStruct((NP, HID), _f32),
            jax.ShapeDtypeStruct((NP, HID), _f32),
        ],
    )(xp, W_gat, att_src, att_dst)


def _m2b_body(degp, dinv128):
    d = degp[0] + degp[1] + 1.0
    dinv128[...] = lax.rsqrt(d)


def _m2b_call(degp):
    return pl.pallas_call(
        _m2b_body,
        grid=(NRB,),
        in_specs=[pl.BlockSpec((NC, RB, HID), lambda i: (0, i, 0))],
        out_specs=pl.BlockSpec((RB, HID), lambda i: (i, 0)),
        out_shape=jax.ShapeDtypeStruct((NP, HID), _f32),
    )(degp)


def _rowmask(i):
    rows = i * RB + lax.broadcasted_iota(_i32, (RB, 1), 0)
    return rows < N


def _m2_body(agg, sp, bg, g, b, out, acc):
    p = pl.program_id(0)
    i = pl.program_id(1)
    s = sp[0] + sp[1]
    s4 = s[:, :H]
    at = agg[...].swapaxes(0, 1)
    y3 = at / (s4[:, :, None] + 1e-16)
    y = y3.reshape(RB, H * HID) + bg[0]

    @pl.when(p == 0)
    def _():
        @pl.when(i == 0)
        def _():
            acc[...] = jnp.zeros_like(acc)
        acc[0] = acc[0] + y.sum(0)
        acc[1] = acc[1] + (y * y).sum(0)

    @pl.when(p == 1)
    def _():
        mean = acc[0] / N
        var = acc[1] / N - mean * mean
        yn = (y - mean) * lax.rsqrt(var + 1e-5) * g[0] + b[0]
        out[...] = jnp.where(_rowmask(i), jnp.maximum(yn, 0.0), 0.0)


def _m2_call(agg, sparts, b_gat, bn1_g, bn1_b):
    D = H * HID
    return pl.pallas_call(
        _m2_body,
        grid=(2, NRB),
        in_specs=[
            pl.BlockSpec((H, RB, HID), lambda p, i: (0, i, 0)),
            pl.BlockSpec((NC, RB, HID), lambda p, i: (0, i, 0)),
            pl.BlockSpec((1, D), lambda p, i: (0, 0)),
            pl.BlockSpec((1, D), lambda p, i: (0, 0)),
            pl.BlockSpec((1, D), lambda p, i: (0, 0)),
        ],
        out_specs=pl.BlockSpec((RB, D), lambda p, i: (i, 0)),
        out_shape=jax.ShapeDtypeStruct((NP, D), _f32),
        scratch_shapes=[pltpu.VMEM((8, D), _f32)],
    )(agg, sparts, b_gat, bn1_g, bn1_b)


def _mm_body(h, w, out):
    out[...] = jnp.dot(h[...], w[0], preferred_element_type=_f32)[None]


def _mm_call(h, wstack, nb, din):
    return pl.pallas_call(
        _mm_body,
        grid=(nb, NRB),
        in_specs=[
            pl.BlockSpec((RB, din), lambda j, r: (r, 0)),
            pl.BlockSpec((1, din, HID), lambda j, r: (j, 0, 0)),
        ],
        out_specs=pl.BlockSpec((1, RB, HID), lambda j, r: (j, r, 0)),
        out_shape=jax.ShapeDtypeStruct((nb, NP, HID), _f32),
    )(h, wstack)


def _m4_body(agg, hp, dinv, bc, g, b, out, acc):
    p = pl.program_id(0)
    i = pl.program_id(1)
    dv = dinv[:, :NE]
    at = agg[...].swapaxes(0, 1)
    ht = hp[...].swapaxes(0, 1)
    y3 = at * dv[:, :, None] + ht * (dv * dv)[:, :, None] + bc[...][None]
    y = y3.reshape(RB, NE * HID)

    @pl.when(p == 0)
    def _():
        @pl.when(i == 0)
        def _():
            acc[...] = jnp.zeros_like(acc)
        acc[0] = acc[0] + y.sum(0)
        acc[1] = acc[1] + (y * y).sum(0)

    @pl.when(p == 1)
    def _():
        mean = acc[0] / N
        var = acc[1] / N - mean * mean
        yn = (y - mean) * lax.rsqrt(var + 1e-5) * g[0] + b[0]
        out[...] = jnp.where(_rowmask(i), jnp.maximum(yn, 0.0), 0.0)


def _m4_call(agg, hp, dinv16, b_c, bn2_g, bn2_b):
    D = NE * HID
    return pl.pallas_call(
        _m4_body,
        grid=(2, NRB),
        in_specs=[
            pl.BlockSpec((NE, RB, HID), lambda p, i: (0, i, 0)),
            pl.BlockSpec((NE, RB, HID), lambda p, i: (0, i, 0)),
            pl.BlockSpec((RB, HID), lambda p, i: (i, 0)),
            pl.BlockSpec((NE, HID), lambda p, i: (0, 0)),
            pl.BlockSpec((1, D), lambda p, i: (0, 0)),
            pl.BlockSpec((1, D), lambda p, i: (0, 0)),
        ],
        out_specs=pl.BlockSpec((RB, D), lambda p, i: (i, 0)),
        out_shape=jax.ShapeDtypeStruct((NP, D), _f32),
        scratch_shapes=[pltpu.VMEM((8, D), _f32)],
    )(agg, hp, dinv16, b_c, bn2_g, bn2_b)


def _m6_body(agg, hp, dinv, bml, g, b, omu, ols, acc):
    p = pl.program_id(0)
    i = pl.program_id(1)
    dv = dinv[:, :NE]
    dvv = jnp.concatenate([dv, dv], axis=1)
    at = agg[...].swapaxes(0, 1)
    ht = hp[...].swapaxes(0, 1)
    y3 = at * dvv[:, :, None] + ht * (dvv * dvv)[:, :, None] + bml[...][None]
    D = NE * HID
    ymu = y3[:, :NE].reshape(RB, D)
    yls = y3[:, NE:].reshape(RB, D)

    @pl.when(p == 0)
    def _():
        @pl.when(i == 0)
        def _():
            acc[...] = jnp.zeros_like(acc)
        acc[0] = acc[0] + ymu.sum(0)
        acc[1] = acc[1] + (ymu * ymu).sum(0)
        acc[2] = acc[2] + yls.sum(0)
        acc[3] = acc[3] + (yls * yls).sum(0)

    @pl.when(p == 1)
    def _():
        m = _rowmask(i)
        mu_m = acc[0] / N
        mu_v = acc[1] / N - mu_m * mu_m
        ls_m = acc[2] / N
        ls_v = acc[3] / N - ls_m * ls_m
        a = (ymu - mu_m) * lax.rsqrt(mu_v + 1e-5) * g[0] + b[0]
        c = (yls - ls_m) * lax.rsqrt(ls_v + 1e-5) * g[0] + b[0]
        omu[...] = jnp.where(m, jnp.maximum(a, 0.0), 0.0)
        ols[...] = jnp.where(m, jnp.maximum(c, 0.0), 0.0)


def _m6_call(agg, hp, dinv16, b_ml, bn2_g, bn2_b):
    D = NE * HID
    return pl.pallas_call(
        _m6_body,
        grid=(2, NRB),
        in_specs=[
            pl.BlockSpec((2 * NE, RB, HID), lambda p, i: (0, i, 0)),
            pl.BlockSpec((2 * NE, RB, HID), lambda p, i: (0, i, 0)),
            pl.BlockSpec((RB, HID), lambda p, i: (i, 0)),
            pl.BlockSpec((2 * NE, HID), lambda p, i: (0, 0)),
            pl.BlockSpec((1, D), lambda p, i: (0, 0)),
            pl.BlockSpec((1, D), lambda p, i: (0, 0)),
        ],
        out_specs=[
            pl.BlockSpec((RB, D), lambda p, i: (i, 0)),
            pl.BlockSpec((RB, D), lambda p, i: (i, 0)),
        ],
        out_shape=[
            jax.ShapeDtypeStruct((NP, D), _f32),
            jax.ShapeDtypeStruct((NP, D), _f32),
        ],
        scratch_shapes=[pltpu.VMEM((8, D), _f32)],
    )(agg, hp, dinv16, b_ml, bn2_g, bn2_b)


EB = 512
NEB = ET // EB


def _m8_body(r00, r01, r10, r11, b0, w1, b1, w2, b2, out):
    wb = w1[...]
    z0 = jnp.maximum(r00[...] + r01[...] + b0[0], 0.0).astype(jnp.bfloat16)
    z1 = jnp.maximum(r10[...] + r11[...] + b0[0], 0.0).astype(jnp.bfloat16)
    t0 = jnp.maximum(jnp.dot(z0, wb, preferred_element_type=_f32) + b1[0], 0.0)
    t1 = jnp.maximum(jnp.dot(z1, wb, preferred_element_type=_f32) + b1[0], 0.0)
    out[...] = jnp.dot(0.5 * (t0 + t1), w2[...], preferred_element_type=_f32) + b2[0]


def _m8_call(r00, r01, r10, r11, b_l0, W_l1, b_l1, W_l2p, b_l2p):
    es = pl.BlockSpec((EB, LH), lambda i: (i, 0))
    return pl.pallas_call(
        _m8_body,
        grid=(NEB,),
        in_specs=[
            es, es, es, es,
            pl.BlockSpec((1, LH), lambda i: (0, 0)),
            pl.BlockSpec((LH, LH), lambda i: (0, 0)),
            pl.BlockSpec((1, LH), lambda i: (0, 0)),
            pl.BlockSpec((LH, HID), lambda i: (0, 0)),
            pl.BlockSpec((1, HID), lambda i: (0, 0)),
        ],
        out_specs=pl.BlockSpec((EB, HID), lambda i: (i, 0)),
        out_shape=jax.ShapeDtypeStruct((ET, HID), _f32),
    )(r00, r01, r10, r11, b_l0, W_l1, b_l1, W_l2p, b_l2p)


def _pad_idx(n_extra):
    return (N + (jnp.arange(n_extra, dtype=_i32) % (NP - N))).astype(_i32)


def kernel(x, edge_index, edge_attr, edge_index_test, W_gat, att_src, att_dst,
           b_gat, bn1_g, bn1_b, bn2_g, bn2_b, W_c, b_c, W_mu, b_mu, W_ls, b_ls,
           W_l0, b_l0, W_l1, b_l1, W_l2, b_l2):
    xp = jnp.pad(x, ((0, NP - N), (0, 0)))
    loop = jnp.arange(N, dtype=_i32)
    srcg = jnp.concatenate([edge_index[0], loop, _pad_idx(EG - E - N)])
    dstg = jnp.concatenate([edge_index[1], loop, _pad_idx(EG - E - N)])
    srcw = jnp.concatenate([edge_index[0], _pad_idx(EW - E)])
    dstw = jnp.concatenate([edge_index[1], _pad_idx(EW - E)])
    st = jnp.concatenate([edge_index_test[0], _pad_idx(ET - E)])
    dt = jnp.concatenate([edge_index_test[1], _pad_idx(ET - E)])
    wflat = jnp.pad(edge_attr, ((0, EW - E), (0, L - NE))).reshape(EW * L)
    z128 = jnp.zeros((NP, HID), _f32)

    W_mlp = jnp.concatenate([W_mu, W_ls], axis=0)
    b_mlp = jnp.concatenate([b_mu, b_ls], axis=0)
    W0ab = W_l0.reshape(2, NE * HID, LH)
    W_l2p = jnp.pad(W_l2, ((0, 0), (0, HID - 4)))
    b_l2p = jnp.pad(b_l2, (0, HID - 4)).reshape(1, HID)
    b_gat2 = b_gat.reshape(1, -1)
    bn1_g2, bn1_b2 = bn1_g.reshape(1, -1), bn1_b.reshape(1, -1)
    bn2_g2, bn2_b2 = bn2_g.reshape(1, -1), bn2_b.reshape(1, -1)
    b_l0_2, b_l1_2 = b_l0.reshape(1, -1), b_l1.reshape(1, -1)

    xlbm, als128, ald128 = _m1_call(xp, W_gat, att_src, att_dst)
    e_sc = _k1a_call(srcg, dstg, als128, ald128)
    sparts = _k2a_call(dstg, e_sc, z128, EG)
    agg1 = _k3_call(srcg, dstg, e_sc, xlbm.reshape(H * NP, HID), z128,
                    nbp=H, ncol=16, etot=EG)
    h1 = _m2_call(agg1.reshape(H, NP, HID), sparts.reshape(NC, NP, HID),
                  b_gat2, bn1_g2, bn1_b2)

    degp = _k2a_call(dstw, wflat, z128, EW)
    dinv128 = _m2b_call(degp.reshape(NC, NP, HID))
    scale = _k2b_call(srcw, wflat, dinv128)

    hp_c = _mm_call(h1, W_c, NE, H * HID)
    agg2 = _k3_call(srcw, dstw, scale, hp_c.reshape(NE * NP, HID), z128,
                    nbp=NE, ncol=16, etot=EW)
    h2 = _m4_call(agg2.reshape(NE, NP, HID), hp_c, dinv128, b_c,
                  bn2_g2, bn2_b2)

    hp_ml = _mm_call(h2, W_mlp, 2 * NE, NE * HID)
    agg3 = _k3_call(srcw, dstw, scale, hp_ml.reshape(2 * NE * NP, HID), z128,
                    nbp=2 * NE, ncol=NE, etot=EW)
    xmu_p, xls_p = _m6_call(agg3.reshape(2 * NE, NP, HID), hp_ml, dinv128,
                            b_mlp, bn2_g2, bn2_b2)

    pq = _mm_call2(xmu_p, W0ab)
    r00, r01, r10, r11 = _k4_call(st, dt, pq.reshape(2 * NP, LH))
    dfull = _m8_call(r00, r01, r10, r11, b_l0_2, W_l1.astype(jnp.bfloat16),
                     b_l1_2, W_l2p, b_l2p)

    return (xmu_p[:N], xls_p[:N], dfull[:E, :4])


def _mm2_body(h, w, out):
    out[...] = jnp.dot(h[...], w[0], preferred_element_type=_f32)[None]


def _mm_call2(h, wstack):
    return pl.pallas_call(
        _mm2_body,
        grid=(2, NRB),
        in_specs=[
            pl.BlockSpec((RB, NE * HID), lambda j, r: (r, 0)),
            pl.BlockSpec((1, NE * HID, LH), lambda j, r: (j, 0, 0)),
        ],
        out_specs=pl.BlockSpec((1, RB, LH), lambda j, r: (j, r, 0)),
        out_shape=jax.ShapeDtypeStruct((2, NP, LH), _f32),
    )(h, wstack)


# speedup vs baseline: 8.4880x; 1.0309x over previous
import functools

import jax
import jax.numpy as jnp
from jax import lax
from jax.experimental import pallas as pl
from jax.experimental.pallas import tpu as pltpu
from jax.experimental.pallas import tpu_sc as plsc

N = 10000
NP = 10240
IN = 256
HID = 128
H = 4
NE = 5
LH = 256
E = 160000

NC, NS, L = 2, 16, 16
NW = NC * NS
CH = 128

EG = 172032
EW = 163840
ET = 163840
RPT = NP // NS

_f32 = jnp.float32
_i32 = jnp.int32


def _mesh():
    return plsc.VectorSubcoreMesh(core_axis_name="c", subcore_axis_name="s",
                                  num_cores=NC, num_subcores=NS)


def _k1a_body(src_r, dst_r, als_r, ald_r, e_out, si, di, av, bv, e16, sem):
    cid = lax.axis_index("c")
    sid = lax.axis_index("s")
    w = sid * NC + cid
    nch = EG // (NW * CH)

    def chunk(ch, _):
        base = w * (EG // NW) + ch * CH
        pltpu.sync_copy(src_r.at[pl.ds(base, CH)], si)
        pltpu.sync_copy(dst_r.at[pl.ds(base, CH)], di)
        c0 = pltpu.async_copy(als_r.at[si], av, sem)
        c1 = pltpu.async_copy(ald_r.at[di], bv, sem)
        c0.wait()
        c1.wait()

        def row(r, c):
            v = av[r, pl.ds(0, L)] + bv[r, pl.ds(0, L)]
            v = jnp.where(v >= 0, v, 0.2 * v)
            e16[pl.ds(r * L, L)] = jnp.exp(v)
            return c
        lax.fori_loop(0, CH, row, 0)
        pltpu.sync_copy(e16, e_out.at[pl.ds(base * L, CH * L)])
        return _
    lax.fori_loop(0, nch, chunk, 0)


def _k1a_call(srcg, dstg, als128, ald128):
    f = pl.kernel(
        _k1a_body,
        out_type=jax.ShapeDtypeStruct((EG * L,), _f32),
        mesh=_mesh(),
        scratch_types=[
            pltpu.VMEM((CH,), _i32), pltpu.VMEM((CH,), _i32),
            pltpu.VMEM((CH, HID), _f32), pltpu.VMEM((CH, HID), _f32),
            pltpu.VMEM((CH * L,), _f32),
            pltpu.SemaphoreType.DMA,
        ],
    )
    return f(srcg, dstg, als128, ald128)


def _k2a_body(dst_r, w_r, z_r, d_out, di, wv, wv128, dacc, *, etot):
    cid = lax.axis_index("c")
    sid = lax.axis_index("s")
    w = sid * NC + cid
    rows0 = sid * RPT
    pltpu.sync_copy(z_r.at[pl.ds(rows0, RPT)], dacc.at[pl.ds(rows0, RPT)])

    def zrow(r, c):
        for j in range(1, HID // L):
            wv128[r, pl.ds(j * L, L)] = jnp.zeros((L,), _f32)
        return c
    lax.fori_loop(0, CH, zrow, 0)
    plsc.subcore_barrier()
    nch = etot // (NW * CH)

    def chunk(ch, _):
        base = w * (etot // NW) + ch * CH
        pltpu.sync_copy(dst_r.at[pl.ds(base, CH)], di)
        pltpu.sync_copy(w_r.at[pl.ds(base * L, CH * L)], wv)

        def row(r, c):
            wv128[r, pl.ds(0, L)] = wv[pl.ds(r * L, L)]
            return c
        lax.fori_loop(0, CH, row, 0)
        pltpu.sync_copy(wv128, dacc.at[di], add=True)
        return _
    lax.fori_loop(0, nch, chunk, 0)
    plsc.subcore_barrier()
    pltpu.sync_copy(dacc.at[pl.ds(rows0, RPT)],
                    d_out.at[pl.ds(cid * NP + rows0, RPT)])


def _k2a_call(dst, wflat, z128, etot):
    body = functools.partial(_k2a_body, etot=etot)
    f = pl.kernel(
        body,
        out_type=jax.ShapeDtypeStruct((NC * NP, HID), _f32),
        mesh=_mesh(),
        scratch_types=[
            pltpu.VMEM((CH,), _i32), pltpu.VMEM((CH * L,), _f32),
            pltpu.VMEM((CH, HID), _f32),
            pltpu.VMEM_SHARED((NP, HID), _f32),
        ],
    )
    return f(dst, wflat, z128)


def _k2b_body(src_r, w_r, dinv_r, s_out, si, wv, dv, sem):
    cid = lax.axis_index("c")
    sid = lax.axis_index("s")
    w = sid * NC + cid
    nch = EW // (NW * CH)

    def chunk(ch, _):
        base = w * (EW // NW) + ch * CH
        pltpu.sync_copy(src_r.at[pl.ds(base, CH)], si)
        pltpu.sync_copy(w_r.at[pl.ds(base * L, CH * L)], wv)
        pltpu.async_copy(dinv_r.at[si], dv, sem).wait()

        def row(r, c):
            wv[pl.ds(r * L, L)] = wv[pl.ds(r * L, L)] * dv[r, pl.ds(0, L)]
            return c
        lax.fori_loop(0, CH, row, 0)
        pltpu.sync_copy(wv, s_out.at[pl.ds(base * L, CH * L)])
        return _
    lax.fori_loop(0, nch, chunk, 0)


def _k2b_call(srcw, wflat, dinv128):
    f = pl.kernel(
        _k2b_body,
        out_type=jax.ShapeDtypeStruct((EW * L,), _f32),
        mesh=_mesh(),
        scratch_types=[
            pltpu.VMEM((CH,), _i32), pltpu.VMEM((CH * L,), _f32),
            pltpu.VMEM((CH, HID), _f32), pltpu.SemaphoreType.DMA,
        ],
    )
    return f(srcw, wflat, dinv128)


def _k3_body(src_r, dst_r, sc_r, tab_r, z_r, out_r,
             si0, si1, di0, di1, gi0, gi1, sv0, sv1, rw0, rw1, acc,
             sem0, sem1, semw0, semw1, *, nbp, ncol, etot):
    cid = lax.axis_index("c")
    sid = lax.axis_index("s")
    rows0 = sid * RPT
    ept = etot // NS
    nch = ept // CH
    sis, dis, gis, svs, rws = [si0, si1], [di0, di1], [gi0, gi1], \
        [sv0, sv1], [rw0, rw1]
    sems = [sem0, sem1]
    semws = [semw0, semw1]

    def stage(c, b, boff, wait_prev):
        if wait_prev is not False:
            @pl.when(wait_prev)
            def _w():
                pltpu.make_async_copy(rws[b], acc.at[dis[b]], semws[b]).wait()
        base = sid * ept + c * CH
        pltpu.sync_copy(src_r.at[pl.ds(base, CH)], sis[b])
        pltpu.sync_copy(dst_r.at[pl.ds(base, CH)], dis[b])
        pltpu.sync_copy(sc_r.at[pl.ds(base * L, CH * L)], svs[b])

        def addi(r, cc):
            gis[b][pl.ds(r * L, L)] = sis[b][pl.ds(r * L, L)] + boff
            return cc
        lax.fori_loop(0, CH // L, addi, 0)
        pltpu.async_copy(tab_r.at[gis[b]], rws[b], sems[b])

    def block(k, carry):
        b = k * NC + cid
        boff = b * NP
        colv = jnp.full((L,), lax.rem(b, jnp.int32(ncol)), _i32)
        pltpu.sync_copy(z_r.at[pl.ds(rows0, RPT)], acc.at[pl.ds(rows0, RPT)])
        plsc.subcore_barrier()
        stage(0, 0, boff, False)

        def chunk2(cc, carry2):
            for bb in range(2):
                c = cc * 2 + bb

                @pl.when(c + 1 < nch)
                def _stage_next():
                    stage(c + 1, 1 - bb, boff, c + 1 >= 2)
                pltpu.make_async_copy(tab_r.at[gis[bb]], rws[bb],
                                      sems[bb]).wait()

                def edge(e4, c2):
                    for u in range(4):
                        e = e4 * 4 + u
                        v = svs[bb][pl.ds(e * L, L)]
                        sp = v[colv]
                        for j in range(HID // L):
                            rws[bb][e, pl.ds(j * L, L)] = \
                                rws[bb][e, pl.ds(j * L, L)] * sp
                    return c2
                lax.fori_loop(0, CH // 4, edge, 0)
                pltpu.async_copy(rws[bb], acc.at[dis[bb]], semws[bb],
                                 add=True)
            return carry2
        lax.fori_loop(0, nch // 2, chunk2, 0)
        for b2 in range(2):
            pltpu.make_async_copy(rws[b2], acc.at[dis[b2]], semws[b2]).wait()
        plsc.subcore_barrier()
        pltpu.sync_copy(acc.at[pl.ds(rows0, RPT)],
                        out_r.at[pl.ds(boff + rows0, RPT)])
        plsc.subcore_barrier()
        return carry
    lax.fori_loop(0, (nbp - cid + NC - 1) // NC, block, 0)


def _k3_call(src, dst, scale, table, z128, *, nbp, ncol, etot):
    body = functools.partial(_k3_body, nbp=nbp, ncol=ncol, etot=etot)
    f = pl.kernel(
        body,
        out_type=jax.ShapeDtypeStruct((nbp * NP, HID), _f32),
        mesh=_mesh(),
        scratch_types=[
            pltpu.VMEM((CH,), _i32), pltpu.VMEM((CH,), _i32),
            pltpu.VMEM((CH,), _i32), pltpu.VMEM((CH,), _i32),
            pltpu.VMEM((CH,), _i32), pltpu.VMEM((CH,), _i32),
            pltpu.VMEM((CH * L,), _f32), pltpu.VMEM((CH * L,), _f32),
            pltpu.VMEM((CH, HID), _f32), pltpu.VMEM((CH, HID), _f32),
            pltpu.VMEM_SHARED((NP, HID), _f32),
            pltpu.SemaphoreType.DMA, pltpu.SemaphoreType.DMA,
            pltpu.SemaphoreType.DMA, pltpu.SemaphoreType.DMA,
        ],
    )
    return f(src, dst, scale, table, z128)


CH4 = 64


def _k4_body(st_r, dt_r, pq_r, r0_out, r1_out,
             si, di, gis, gid, av, bv, cv, dv, semg, semw):
    cid = lax.axis_index("c")
    sid = lax.axis_index("s")
    w = sid * NC + cid
    nch = ET // (NW * CH4)

    def chunk(ch, _):
        base = w * (ET // NW) + ch * CH4
        pltpu.sync_copy(st_r.at[pl.ds(base, CH4)], si)
        pltpu.sync_copy(dt_r.at[pl.ds(base, CH4)], di)

        def addi(r, c):
            gis[pl.ds(r * L, L)] = si[pl.ds(r * L, L)] + NP
            gid[pl.ds(r * L, L)] = di[pl.ds(r * L, L)] + NP
            return c
        lax.fori_loop(0, CH4 // L, addi, 0)
        c0 = pltpu.async_copy(pq_r.at[si], av, semg)
        c1 = pltpu.async_copy(pq_r.at[gid], bv, semg)
        c2 = pltpu.async_copy(pq_r.at[di], cv, semg)
        c3 = pltpu.async_copy(pq_r.at[gis], dv, semg)
        c0.wait(); c1.wait(); c2.wait(); c3.wait()

        def row(r, c):
            for j in range(LH // L):
                av[r, pl.ds(j * L, L)] = av[r, pl.ds(j * L, L)] + \
                    bv[r, pl.ds(j * L, L)]
                cv[r, pl.ds(j * L, L)] = cv[r, pl.ds(j * L, L)] + \
                    dv[r, pl.ds(j * L, L)]
            return c
        lax.fori_loop(0, CH4, row, 0)
        w0 = pltpu.async_copy(av, r0_out.at[pl.ds(base, CH4)], semw)
        w1 = pltpu.async_copy(cv, r1_out.at[pl.ds(base, CH4)], semw)
        w0.wait(); w1.wait()
        return _
    lax.fori_loop(0, nch, chunk, 0)


def _k4_call(st, dt, pq):
    o = jax.ShapeDtypeStruct((ET, LH), _f32)
    f = pl.kernel(
        _k4_body,
        out_type=(o, o),
        mesh=_mesh(),
        scratch_types=[
            pltpu.VMEM((CH4,), _i32), pltpu.VMEM((CH4,), _i32),
            pltpu.VMEM((CH4,), _i32), pltpu.VMEM((CH4,), _i32),
            pltpu.VMEM((CH4, LH), _f32), pltpu.VMEM((CH4, LH), _f32),
            pltpu.VMEM((CH4, LH), _f32), pltpu.VMEM((CH4, LH), _f32),
            pltpu.SemaphoreType.DMA, pltpu.SemaphoreType.DMA,
        ],
    )
    return f(st, dt, pq)


RB = 256
NRB = NP // RB


def _m1_body(xp, wg, ats, atd, xlbm, als128, ald128):
    a = jnp.dot(xp[...], wg[...], preferred_element_type=_f32)
    a3 = a.reshape(RB, H, HID)
    xlbm[...] = a3.swapaxes(0, 1)
    als = (a3 * ats[...][None]).sum(-1)
    ald = (a3 * atd[...][None]).sum(-1)
    z = jnp.zeros((RB, HID - H), _f32)
    als128[...] = jnp.concatenate([als, z], axis=1)
    ald128[...] = jnp.concatenate([ald, z], axis=1)


def _m1_call(xp, W_gat, att_src, att_dst):
    return pl.pallas_call(
        _m1_body,
        grid=(NRB,),
        in_specs=[
            pl.BlockSpec((RB, IN), lambda i: (i, 0)),
            pl.BlockSpec((IN, H * HID), lambda i: (0, 0)),
            pl.BlockSpec((H, HID), lambda i: (0, 0)),
            pl.BlockSpec((H, HID), lambda i: (0, 0)),
        ],
        out_specs=[
            pl.BlockSpec((H, RB, HID), lambda i: (0, i, 0)),
            pl.BlockSpec((RB, HID), lambda i: (i, 0)),
            pl.BlockSpec((RB, HID), lambda i: (i, 0)),
        ],
        out_shape=[
            jax.ShapeDtypeStruct((H, NP, HID), _f32),
            jax.ShapeDtypeStruct((NP, HID), _f32),
            jax.ShapeDtypeStruct((NP, HID), _f32),
        ],
    )(xp, W_gat, att_src, att_dst)


def _m2b_body(degp, dinv128):
    d = degp[0] + degp[1] + 1.0
    dinv128[...] = lax.rsqrt(d)


def _m2b_call(degp):
    return pl.pallas_call(
        _m2b_body,
        grid=(NRB,),
        in_specs=[pl.BlockSpec((NC, RB, HID), lambda i: (0, i, 0))],
        out_specs=pl.BlockSpec((RB, HID), lambda i: (i, 0)),
        out_shape=jax.ShapeDtypeStruct((NP, HID), _f32),
    )(degp)


def _rowmask(i):
    rows = i * RB + lax.broadcasted_iota(_i32, (RB, 1), 0)
    return rows < N


def _m2_body(agg, sp, bg, g, b, out, acc):
    p = pl.program_id(0)
    i = pl.program_id(1)
    s = sp[0] + sp[1]
    s4 = s[:, :H]
    at = agg[...].swapaxes(0, 1)
    y3 = at / (s4[:, :, None] + 1e-16)
    y = y3.reshape(RB, H * HID) + bg[0]

    @pl.when(p == 0)
    def _():
        @pl.when(i == 0)
        def _():
            acc[...] = jnp.zeros_like(acc)
        acc[0] = acc[0] + y.sum(0)
        acc[1] = acc[1] + (y * y).sum(0)

    @pl.when(p == 1)
    def _():
        mean = acc[0] / N
        var = acc[1] / N - mean * mean
        yn = (y - mean) * lax.rsqrt(var + 1e-5) * g[0] + b[0]
        out[...] = jnp.where(_rowmask(i), jnp.maximum(yn, 0.0), 0.0)


def _m2_call(agg, sparts, b_gat, bn1_g, bn1_b):
    D = H * HID
    return pl.pallas_call(
        _m2_body,
        grid=(2, NRB),
        in_specs=[
            pl.BlockSpec((H, RB, HID), lambda p, i: (0, i, 0)),
            pl.BlockSpec((NC, RB, HID), lambda p, i: (0, i, 0)),
            pl.BlockSpec((1, D), lambda p, i: (0, 0)),
            pl.BlockSpec((1, D), lambda p, i: (0, 0)),
            pl.BlockSpec((1, D), lambda p, i: (0, 0)),
        ],
        out_specs=pl.BlockSpec((RB, D), lambda p, i: (i, 0)),
        out_shape=jax.ShapeDtypeStruct((NP, D), _f32),
        scratch_shapes=[pltpu.VMEM((8, D), _f32)],
    )(agg, sparts, b_gat, bn1_g, bn1_b)


def _mm_body(h, w, out):
    out[...] = jnp.dot(h[...], w[0], preferred_element_type=_f32)[None]


def _mm_call(h, wstack, nb, din):
    return pl.pallas_call(
        _mm_body,
        grid=(nb, NRB),
        in_specs=[
            pl.BlockSpec((RB, din), lambda j, r: (r, 0)),
            pl.BlockSpec((1, din, HID), lambda j, r: (j, 0, 0)),
        ],
        out_specs=pl.BlockSpec((1, RB, HID), lambda j, r: (j, r, 0)),
        out_shape=jax.ShapeDtypeStruct((nb, NP, HID), _f32),
    )(h, wstack)


def _m4_body(agg, hp, dinv, bc, g, b, out, acc):
    p = pl.program_id(0)
    i = pl.program_id(1)
    dv = dinv[:, :NE]
    at = agg[...].swapaxes(0, 1)
    ht = hp[...].swapaxes(0, 1)
    y3 = at * dv[:, :, None] + ht * (dv * dv)[:, :, None] + bc[...][None]
    y = y3.reshape(RB, NE * HID)

    @pl.when(p == 0)
    def _():
        @pl.when(i == 0)
        def _():
            acc[...] = jnp.zeros_like(acc)
        acc[0] = acc[0] + y.sum(0)
        acc[1] = acc[1] + (y * y).sum(0)

    @pl.when(p == 1)
    def _():
        mean = acc[0] / N
        var = acc[1] / N - mean * mean
        yn = (y - mean) * lax.rsqrt(var + 1e-5) * g[0] + b[0]
        out[...] = jnp.where(_rowmask(i), jnp.maximum(yn, 0.0), 0.0)


def _m4_call(agg, hp, dinv16, b_c, bn2_g, bn2_b):
    D = NE * HID
    return pl.pallas_call(
        _m4_body,
        grid=(2, NRB),
        in_specs=[
            pl.BlockSpec((NE, RB, HID), lambda p, i: (0, i, 0)),
            pl.BlockSpec((NE, RB, HID), lambda p, i: (0, i, 0)),
            pl.BlockSpec((RB, HID), lambda p, i: (i, 0)),
            pl.BlockSpec((NE, HID), lambda p, i: (0, 0)),
            pl.BlockSpec((1, D), lambda p, i: (0, 0)),
            pl.BlockSpec((1, D), lambda p, i: (0, 0)),
        ],
        out_specs=pl.BlockSpec((RB, D), lambda p, i: (i, 0)),
        out_shape=jax.ShapeDtypeStruct((NP, D), _f32),
        scratch_shapes=[pltpu.VMEM((8, D), _f32)],
    )(agg, hp, dinv16, b_c, bn2_g, bn2_b)


def _m6_body(agg, hp, dinv, bml, g, b, omu, ols, acc):
    p = pl.program_id(0)
    i = pl.program_id(1)
    dv = dinv[:, :NE]
    dvv = jnp.concatenate([dv, dv], axis=1)
    at = agg[...].swapaxes(0, 1)
    ht = hp[...].swapaxes(0, 1)
    y3 = at * dvv[:, :, None] + ht * (dvv * dvv)[:, :, None] + bml[...][None]
    D = NE * HID
    ymu = y3[:, :NE].reshape(RB, D)
    yls = y3[:, NE:].reshape(RB, D)

    @pl.when(p == 0)
    def _():
        @pl.when(i == 0)
        def _():
            acc[...] = jnp.zeros_like(acc)
        acc[0] = acc[0] + ymu.sum(0)
        acc[1] = acc[1] + (ymu * ymu).sum(0)
        acc[2] = acc[2] + yls.sum(0)
        acc[3] = acc[3] + (yls * yls).sum(0)

    @pl.when(p == 1)
    def _():
        m = _rowmask(i)
        mu_m = acc[0] / N
        mu_v = acc[1] / N - mu_m * mu_m
        ls_m = acc[2] / N
        ls_v = acc[3] / N - ls_m * ls_m
        a = (ymu - mu_m) * lax.rsqrt(mu_v + 1e-5) * g[0] + b[0]
        c = (yls - ls_m) * lax.rsqrt(ls_v + 1e-5) * g[0] + b[0]
        omu[...] = jnp.where(m, jnp.maximum(a, 0.0), 0.0)
        ols[...] = jnp.where(m, jnp.maximum(c, 0.0), 0.0)


def _m6_call(agg, hp, dinv16, b_ml, bn2_g, bn2_b):
    D = NE * HID
    return pl.pallas_call(
        _m6_body,
        grid=(2, NRB),
        in_specs=[
            pl.BlockSpec((2 * NE, RB, HID), lambda p, i: (0, i, 0)),
            pl.BlockSpec((2 * NE, RB, HID), lambda p, i: (0, i, 0)),
            pl.BlockSpec((RB, HID), lambda p, i: (i, 0)),
            pl.BlockSpec((2 * NE, HID), lambda p, i: (0, 0)),
            pl.BlockSpec((1, D), lambda p, i: (0, 0)),
            pl.BlockSpec((1, D), lambda p, i: (0, 0)),
        ],
        out_specs=[
            pl.BlockSpec((RB, D), lambda p, i: (i, 0)),
            pl.BlockSpec((RB, D), lambda p, i: (i, 0)),
        ],
        out_shape=[
            jax.ShapeDtypeStruct((NP, D), _f32),
            jax.ShapeDtypeStruct((NP, D), _f32),
        ],
        scratch_shapes=[pltpu.VMEM((8, D), _f32)],
    )(agg, hp, dinv16, b_ml, bn2_g, bn2_b)


EB = 512
NEB = ET // EB


def _m8_body(r0, r1, b0, w1, b1, w2, b2, out):
    wb = w1[...]
    z0 = jnp.maximum(r0[...] + b0[0], 0.0).astype(jnp.bfloat16)
    z1 = jnp.maximum(r1[...] + b0[0], 0.0).astype(jnp.bfloat16)
    t0 = jnp.maximum(jnp.dot(z0, wb, preferred_element_type=_f32) + b1[0], 0.0)
    t1 = jnp.maximum(jnp.dot(z1, wb, preferred_element_type=_f32) + b1[0], 0.0)
    out[...] = jnp.dot(0.5 * (t0 + t1), w2[...], preferred_element_type=_f32) + b2[0]


def _m8_call(r0, r1, b_l0, W_l1, b_l1, W_l2p, b_l2p):
    es = pl.BlockSpec((EB, LH), lambda i: (i, 0))
    return pl.pallas_call(
        _m8_body,
        grid=(NEB,),
        in_specs=[
            es, es,
            pl.BlockSpec((1, LH), lambda i: (0, 0)),
            pl.BlockSpec((LH, LH), lambda i: (0, 0)),
            pl.BlockSpec((1, LH), lambda i: (0, 0)),
            pl.BlockSpec((LH, HID), lambda i: (0, 0)),
            pl.BlockSpec((1, HID), lambda i: (0, 0)),
        ],
        out_specs=pl.BlockSpec((EB, HID), lambda i: (i, 0)),
        out_shape=jax.ShapeDtypeStruct((ET, HID), _f32),
    )(r0, r1, b_l0, W_l1, b_l1, W_l2p, b_l2p)


def _pad_idx(n_extra):
    return (N + (jnp.arange(n_extra, dtype=_i32) % (NP - N))).astype(_i32)


def kernel(x, edge_index, edge_attr, edge_index_test, W_gat, att_src, att_dst,
           b_gat, bn1_g, bn1_b, bn2_g, bn2_b, W_c, b_c, W_mu, b_mu, W_ls, b_ls,
           W_l0, b_l0, W_l1, b_l1, W_l2, b_l2):
    xp = jnp.pad(x, ((0, NP - N), (0, 0)))
    loop = jnp.arange(N, dtype=_i32)
    srcg = jnp.concatenate([edge_index[0], loop, _pad_idx(EG - E - N)])
    dstg = jnp.concatenate([edge_index[1], loop, _pad_idx(EG - E - N)])
    srcw = jnp.concatenate([edge_index[0], _pad_idx(EW - E)])
    dstw = jnp.concatenate([edge_index[1], _pad_idx(EW - E)])
    st = jnp.concatenate([edge_index_test[0], _pad_idx(ET - E)])
    dt = jnp.concatenate([edge_index_test[1], _pad_idx(ET - E)])
    wflat = jnp.pad(edge_attr, ((0, EW - E), (0, L - NE))).reshape(EW * L)
    z128 = jnp.zeros((NP, HID), _f32)

    W_mlp = jnp.concatenate([W_mu, W_ls], axis=0)
    b_mlp = jnp.concatenate([b_mu, b_ls], axis=0)
    W0ab = W_l0.reshape(2, NE * HID, LH)
    W_l2p = jnp.pad(W_l2, ((0, 0), (0, HID - 4)))
    b_l2p = jnp.pad(b_l2, (0, HID - 4)).reshape(1, HID)
    b_gat2 = b_gat.reshape(1, -1)
    bn1_g2, bn1_b2 = bn1_g.reshape(1, -1), bn1_b.reshape(1, -1)
    bn2_g2, bn2_b2 = bn2_g.reshape(1, -1), bn2_b.reshape(1, -1)
    b_l0_2, b_l1_2 = b_l0.reshape(1, -1), b_l1.reshape(1, -1)

    xlbm, als128, ald128 = _m1_call(xp, W_gat, att_src, att_dst)
    e_sc = _k1a_call(srcg, dstg, als128, ald128)
    sparts = _k2a_call(dstg, e_sc, z128, EG)
    agg1 = _k3_call(srcg, dstg, e_sc, xlbm.reshape(H * NP, HID), z128,
                    nbp=H, ncol=16, etot=EG)
    h1 = _m2_call(agg1.reshape(H, NP, HID), sparts.reshape(NC, NP, HID),
                  b_gat2, bn1_g2, bn1_b2)

    degp = _k2a_call(dstw, wflat, z128, EW)
    dinv128 = _m2b_call(degp.reshape(NC, NP, HID))
    scale = _k2b_call(srcw, wflat, dinv128)

    hp_c = _mm_call(h1, W_c, NE, H * HID)
    agg2 = _k3_call(srcw, dstw, scale, hp_c.reshape(NE * NP, HID), z128,
                    nbp=NE, ncol=16, etot=EW)
    h2 = _m4_call(agg2.reshape(NE, NP, HID), hp_c, dinv128, b_c,
                  bn2_g2, bn2_b2)

    hp_ml = _mm_call(h2, W_mlp, 2 * NE, NE * HID)
    agg3 = _k3_call(srcw, dstw, scale, hp_ml.reshape(2 * NE * NP, HID), z128,
                    nbp=2 * NE, ncol=NE, etot=EW)
    xmu_p, xls_p = _m6_call(agg3.reshape(2 * NE, NP, HID), hp_ml, dinv128,
                            b_mlp, bn2_g2, bn2_b2)

    pq = _mm_call2(xmu_p, W0ab)
    r0, r1 = _k4_call(st, dt, pq.reshape(2 * NP, LH))
    dfull = _m8_call(r0, r1, b_l0_2, W_l1.astype(jnp.bfloat16),
                     b_l1_2, W_l2p, b_l2p)

    return (xmu_p[:N], xls_p[:N], dfull[:E, :4])


def _mm2_body(h, w, out):
    out[...] = jnp.dot(h[...], w[0], preferred_element_type=_f32)[None]


def _mm_call2(h, wstack):
    return pl.pallas_call(
        _mm2_body,
        grid=(2, NRB),
        in_specs=[
            pl.BlockSpec((RB, NE * HID), lambda j, r: (r, 0)),
            pl.BlockSpec((1, NE * HID, LH), lambda j, r: (j, 0, 0)),
        ],
        out_specs=pl.BlockSpec((1, RB, LH), lambda j, r: (j, r, 0)),
        out_shape=jax.ShapeDtypeStruct((2, NP, LH), _f32),
    )(h, wstack)


# speedup vs baseline: 8.4960x; 1.0010x over previous
import functools

import jax
import jax.numpy as jnp
from jax import lax
from jax.experimental import pallas as pl
from jax.experimental.pallas import tpu as pltpu
from jax.experimental.pallas import tpu_sc as plsc

N = 10000
NP = 10240
IN = 256
HID = 128
H = 4
NE = 5
LH = 256
E = 160000

NC, NS, L = 2, 16, 16
NW = NC * NS
CH = 128

EG = 172032
EW = 163840
ET = 163840
RPT = NP // NS

_f32 = jnp.float32
_i32 = jnp.int32


def _mesh():
    return plsc.VectorSubcoreMesh(core_axis_name="c", subcore_axis_name="s",
                                  num_cores=NC, num_subcores=NS)


def _k1a_body(src_r, dst_r, als_r, ald_r, e_out, si, di, av, bv, e16, sem):
    cid = lax.axis_index("c")
    sid = lax.axis_index("s")
    w = sid * NC + cid
    nch = EG // (NW * CH)

    def chunk(ch, _):
        base = w * (EG // NW) + ch * CH
        pltpu.sync_copy(src_r.at[pl.ds(base, CH)], si)
        pltpu.sync_copy(dst_r.at[pl.ds(base, CH)], di)
        c0 = pltpu.async_copy(als_r.at[si], av, sem)
        c1 = pltpu.async_copy(ald_r.at[di], bv, sem)
        c0.wait()
        c1.wait()

        def row(r, c):
            v = av[r, pl.ds(0, L)] + bv[r, pl.ds(0, L)]
            v = jnp.where(v >= 0, v, 0.2 * v)
            e16[pl.ds(r * L, L)] = jnp.exp(v)
            return c
        lax.fori_loop(0, CH, row, 0)
        pltpu.sync_copy(e16, e_out.at[pl.ds(base * L, CH * L)])
        return _
    lax.fori_loop(0, nch, chunk, 0)


def _k1a_call(srcg, dstg, als128, ald128):
    f = pl.kernel(
        _k1a_body,
        out_type=jax.ShapeDtypeStruct((EG * L,), _f32),
        mesh=_mesh(),
        scratch_types=[
            pltpu.VMEM((CH,), _i32), pltpu.VMEM((CH,), _i32),
            pltpu.VMEM((CH, HID), _f32), pltpu.VMEM((CH, HID), _f32),
            pltpu.VMEM((CH * L,), _f32),
            pltpu.SemaphoreType.DMA,
        ],
    )
    return f(srcg, dstg, als128, ald128)


def _k2a_body(dst_r, w_r, z_r, d_out, di, wv, wv128, dacc, *, etot):
    cid = lax.axis_index("c")
    sid = lax.axis_index("s")
    w = sid * NC + cid
    rows0 = sid * RPT
    pltpu.sync_copy(z_r.at[pl.ds(rows0, RPT)], dacc.at[pl.ds(rows0, RPT)])

    def zrow(r, c):
        for j in range(1, HID // L):
            wv128[r, pl.ds(j * L, L)] = jnp.zeros((L,), _f32)
        return c
    lax.fori_loop(0, CH, zrow, 0)
    plsc.subcore_barrier()
    nch = etot // (NW * CH)

    def chunk(ch, _):
        base = w * (etot // NW) + ch * CH
        pltpu.sync_copy(dst_r.at[pl.ds(base, CH)], di)
        pltpu.sync_copy(w_r.at[pl.ds(base * L, CH * L)], wv)

        def row(r, c):
            wv128[r, pl.ds(0, L)] = wv[pl.ds(r * L, L)]
            return c
        lax.fori_loop(0, CH, row, 0)
        pltpu.sync_copy(wv128, dacc.at[di], add=True)
        return _
    lax.fori_loop(0, nch, chunk, 0)
    plsc.subcore_barrier()
    pltpu.sync_copy(dacc.at[pl.ds(rows0, RPT)],
                    d_out.at[pl.ds(cid * NP + rows0, RPT)])


def _k2a_call(dst, wflat, z128, etot):
    body = functools.partial(_k2a_body, etot=etot)
    f = pl.kernel(
        body,
        out_type=jax.ShapeDtypeStruct((NC * NP, HID), _f32),
        mesh=_mesh(),
        scratch_types=[
            pltpu.VMEM((CH,), _i32), pltpu.VMEM((CH * L,), _f32),
            pltpu.VMEM((CH, HID), _f32),
            pltpu.VMEM_SHARED((NP, HID), _f32),
        ],
    )
    return f(dst, wflat, z128)


def _k2b_body(src_r, w_r, dinv_r, s_out, si, wv, dv, sem):
    cid = lax.axis_index("c")
    sid = lax.axis_index("s")
    w = sid * NC + cid
    nch = EW // (NW * CH)

    def chunk(ch, _):
        base = w * (EW // NW) + ch * CH
        pltpu.sync_copy(src_r.at[pl.ds(base, CH)], si)
        pltpu.sync_copy(w_r.at[pl.ds(base * L, CH * L)], wv)
        pltpu.async_copy(dinv_r.at[si], dv, sem).wait()

        def row(r, c):
            wv[pl.ds(r * L, L)] = wv[pl.ds(r * L, L)] * dv[r, pl.ds(0, L)]
            return c
        lax.fori_loop(0, CH, row, 0)
        pltpu.sync_copy(wv, s_out.at[pl.ds(base * L, CH * L)])
        return _
    lax.fori_loop(0, nch, chunk, 0)


def _k2b_call(srcw, wflat, dinv128):
    f = pl.kernel(
        _k2b_body,
        out_type=jax.ShapeDtypeStruct((EW * L,), _f32),
        mesh=_mesh(),
        scratch_types=[
            pltpu.VMEM((CH,), _i32), pltpu.VMEM((CH * L,), _f32),
            pltpu.VMEM((CH, HID), _f32), pltpu.SemaphoreType.DMA,
        ],
    )
    return f(srcw, wflat, dinv128)


def _k3_body(src_r, dst_r, sc_r, tab_r, z_r, out_r,
             si0, si1, di0, di1, gi0, gi1, sv0, sv1, rw0, rw1, acc,
             sem0, sem1, semw0, semw1, *, nbp, ncol, etot):
    cid = lax.axis_index("c")
    sid = lax.axis_index("s")
    rows0 = sid * RPT
    ept = etot // NS
    nch = ept // CH
    sis, dis, gis, svs, rws = [si0, si1], [di0, di1], [gi0, gi1], \
        [sv0, sv1], [rw0, rw1]
    sems = [sem0, sem1]
    semws = [semw0, semw1]

    def stage(c, b, boff, wait_prev):
        if wait_prev is not False:
            @pl.when(wait_prev)
            def _w():
                pltpu.make_async_copy(rws[b], acc.at[dis[b]], semws[b]).wait()
        base = c
        pltpu.sync_copy(src_r.at[pl.ds(base, CH)], sis[b])
        pltpu.sync_copy(dst_r.at[pl.ds(base, CH)], dis[b])
        pltpu.sync_copy(sc_r.at[pl.ds(base * L, CH * L)], svs[b])

        def addi(r, cc):
            gis[b][pl.ds(r * L, L)] = sis[b][pl.ds(r * L, L)] + boff
            return cc
        lax.fori_loop(0, CH // L, addi, 0)
        pltpu.async_copy(tab_r.at[gis[b]], rws[b], sems[b])

    odd = nbp % 2 == 1
    trips = (nbp + NC - 1) // NC

    def block(k, carry):
        if odd:
            is_l = k == trips - 1
            b = jnp.where(is_l, nbp - 1, k * NC + cid)
            nch_eff = jnp.where(is_l, nch // 2, nch)
            ebase = sid * ept + jnp.where(is_l, cid * (ept // 2), 0)
            ooff = (b + jnp.where(is_l, cid, 0)) * NP
        else:
            b = k * NC + cid
            nch_eff = nch
            ebase = sid * ept
            ooff = b * NP
        boff = b * NP
        colv = jnp.full((L,), lax.rem(b, jnp.int32(ncol)), _i32)
        pltpu.sync_copy(z_r.at[pl.ds(rows0, RPT)], acc.at[pl.ds(rows0, RPT)])
        plsc.subcore_barrier()
        stage(ebase, 0, boff, False)

        def chunk2(cc, carry2):
            for bb in range(2):
                c = cc * 2 + bb

                @pl.when(c + 1 < nch_eff)
                def _stage_next():
                    stage(ebase + (c + 1) * CH, 1 - bb, boff, c + 1 >= 2)
                pltpu.make_async_copy(tab_r.at[gis[bb]], rws[bb],
                                      sems[bb]).wait()

                def edge(e4, c2):
                    for u in range(4):
                        e = e4 * 4 + u
                        v = svs[bb][pl.ds(e * L, L)]
                        sp = v[colv]
                        for j in range(HID // L):
                            rws[bb][e, pl.ds(j * L, L)] = \
                                rws[bb][e, pl.ds(j * L, L)] * sp
                    return c2
                lax.fori_loop(0, CH // 4, edge, 0)
                pltpu.async_copy(rws[bb], acc.at[dis[bb]], semws[bb],
                                 add=True)
            return carry2
        lax.fori_loop(0, nch_eff // 2, chunk2, 0)
        for b2 in range(2):
            pltpu.make_async_copy(rws[b2], acc.at[dis[b2]], semws[b2]).wait()
        plsc.subcore_barrier()
        pltpu.sync_copy(acc.at[pl.ds(rows0, RPT)],
                        out_r.at[pl.ds(ooff + rows0, RPT)])
        plsc.subcore_barrier()
        return carry
    lax.fori_loop(0, trips, block, 0)


def _k3_call(src, dst, scale, table, z128, *, nbp, ncol, etot):
    body = functools.partial(_k3_body, nbp=nbp, ncol=ncol, etot=etot)
    nbo = nbp + (nbp % 2)
    f = pl.kernel(
        body,
        out_type=jax.ShapeDtypeStruct((nbo * NP, HID), _f32),
        mesh=_mesh(),
        scratch_types=[
            pltpu.VMEM((CH,), _i32), pltpu.VMEM((CH,), _i32),
            pltpu.VMEM((CH,), _i32), pltpu.VMEM((CH,), _i32),
            pltpu.VMEM((CH,), _i32), pltpu.VMEM((CH,), _i32),
            pltpu.VMEM((CH * L,), _f32), pltpu.VMEM((CH * L,), _f32),
            pltpu.VMEM((CH, HID), _f32), pltpu.VMEM((CH, HID), _f32),
            pltpu.VMEM_SHARED((NP, HID), _f32),
            pltpu.SemaphoreType.DMA, pltpu.SemaphoreType.DMA,
            pltpu.SemaphoreType.DMA, pltpu.SemaphoreType.DMA,
        ],
    )
    return f(src, dst, scale, table, z128)


CH4 = 64


def _k4_body(st_r, dt_r, pq_r, r0_out, r1_out,
             si, di, gis, gid, av, bv, cv, dv, semg, semw):
    cid = lax.axis_index("c")
    sid = lax.axis_index("s")
    w = sid * NC + cid
    nch = ET // (NW * CH4)

    def chunk(ch, _):
        base = w * (ET // NW) + ch * CH4
        pltpu.sync_copy(st_r.at[pl.ds(base, CH4)], si)
        pltpu.sync_copy(dt_r.at[pl.ds(base, CH4)], di)

        def addi(r, c):
            gis[pl.ds(r * L, L)] = si[pl.ds(r * L, L)] + NP
            gid[pl.ds(r * L, L)] = di[pl.ds(r * L, L)] + NP
            return c
        lax.fori_loop(0, CH4 // L, addi, 0)
        c0 = pltpu.async_copy(pq_r.at[si], av, semg)
        c1 = pltpu.async_copy(pq_r.at[gid], bv, semg)
        c2 = pltpu.async_copy(pq_r.at[di], cv, semg)
        c3 = pltpu.async_copy(pq_r.at[gis], dv, semg)
        c0.wait(); c1.wait(); c2.wait(); c3.wait()

        def row(r, c):
            for j in range(LH // L):
                av[r, pl.ds(j * L, L)] = av[r, pl.ds(j * L, L)] + \
                    bv[r, pl.ds(j * L, L)]
                cv[r, pl.ds(j * L, L)] = cv[r, pl.ds(j * L, L)] + \
                    dv[r, pl.ds(j * L, L)]
            return c
        lax.fori_loop(0, CH4, row, 0)
        w0 = pltpu.async_copy(av, r0_out.at[pl.ds(base, CH4)], semw)
        w1 = pltpu.async_copy(cv, r1_out.at[pl.ds(base, CH4)], semw)
        w0.wait(); w1.wait()
        return _
    lax.fori_loop(0, nch, chunk, 0)


def _k4_call(st, dt, pq):
    o = jax.ShapeDtypeStruct((ET, LH), _f32)
    f = pl.kernel(
        _k4_body,
        out_type=(o, o),
        mesh=_mesh(),
        scratch_types=[
            pltpu.VMEM((CH4,), _i32), pltpu.VMEM((CH4,), _i32),
            pltpu.VMEM((CH4,), _i32), pltpu.VMEM((CH4,), _i32),
            pltpu.VMEM((CH4, LH), _f32), pltpu.VMEM((CH4, LH), _f32),
            pltpu.VMEM((CH4, LH), _f32), pltpu.VMEM((CH4, LH), _f32),
            pltpu.SemaphoreType.DMA, pltpu.SemaphoreType.DMA,
        ],
    )
    return f(st, dt, pq)


RB = 256
NRB = NP // RB


def _m1_body(xp, wg, ats, atd, xlbm, als128, ald128):
    a = jnp.dot(xp[...], wg[...], preferred_element_type=_f32)
    a3 = a.reshape(RB, H, HID)
    xlbm[...] = a3.swapaxes(0, 1)
    als = (a3 * ats[...][None]).sum(-1)
    ald = (a3 * atd[...][None]).sum(-1)
    z = jnp.zeros((RB, HID - H), _f32)
    als128[...] = jnp.concatenate([als, z], axis=1)
    ald128[...] = jnp.concatenate([ald, z], axis=1)


def _m1_call(xp, W_gat, att_src, att_dst):
    return pl.pallas_call(
        _m1_body,
        grid=(NRB,),
        in_specs=[
            pl.BlockSpec((RB, IN), lambda i: (i, 0)),
            pl.BlockSpec((IN, H * HID), lambda i: (0, 0)),
            pl.BlockSpec((H, HID), lambda i: (0, 0)),
            pl.BlockSpec((H, HID), lambda i: (0, 0)),
        ],
        out_specs=[
            pl.BlockSpec((H, RB, HID), lambda i: (0, i, 0)),
            pl.BlockSpec((RB, HID), lambda i: (i, 0)),
            pl.BlockSpec((RB, HID), lambda i: (i, 0)),
        ],
        out_shape=[
            jax.ShapeDtypeStruct((H, NP, HID), _f32),
            jax.ShapeDtypeStruct((NP, HID), _f32),
            jax.ShapeDtypeStruct((NP, HID), _f32),
        ],
    )(xp, W_gat, att_src, att_dst)


def _m2b_body(degp, dinv128):
    d = degp[0] + degp[1] + 1.0
    dinv128[...] = lax.rsqrt(d)


def _m2b_call(degp):
    return pl.pallas_call(
        _m2b_body,
        grid=(NRB,),
        in_specs=[pl.BlockSpec((NC, RB, HID), lambda i: (0, i, 0))],
        out_specs=pl.BlockSpec((RB, HID), lambda i: (i, 0)),
        out_shape=jax.ShapeDtypeStruct((NP, HID), _f32),
    )(degp)


def _rowmask(i):
    rows = i * RB + lax.broadcasted_iota(_i32, (RB, 1), 0)
    return rows < N


def _m2_body(agg, sp, bg, g, b, out, acc):
    p = pl.program_id(0)
    i = pl.program_id(1)
    s = sp[0] + sp[1]
    s4 = s[:, :H]
    at = agg[...].swapaxes(0, 1)
    y3 = at / (s4[:, :, None] + 1e-16)
    y = y3.reshape(RB, H * HID) + bg[0]

    @pl.when(p == 0)
    def _():
        @pl.when(i == 0)
        def _():
            acc[...] = jnp.zeros_like(acc)
        acc[0] = acc[0] + y.sum(0)
        acc[1] = acc[1] + (y * y).sum(0)

    @pl.when(p == 1)
    def _():
        mean = acc[0] / N
        var = acc[1] / N - mean * mean
        yn = (y - mean) * lax.rsqrt(var + 1e-5) * g[0] + b[0]
        out[...] = jnp.where(_rowmask(i), jnp.maximum(yn, 0.0), 0.0)


def _m2_call(agg, sparts, b_gat, bn1_g, bn1_b):
    D = H * HID
    return pl.pallas_call(
        _m2_body,
        grid=(2, NRB),
        in_specs=[
            pl.BlockSpec((H, RB, HID), lambda p, i: (0, i, 0)),
            pl.BlockSpec((NC, RB, HID), lambda p, i: (0, i, 0)),
            pl.BlockSpec((1, D), lambda p, i: (0, 0)),
            pl.BlockSpec((1, D), lambda p, i: (0, 0)),
            pl.BlockSpec((1, D), lambda p, i: (0, 0)),
        ],
        out_specs=pl.BlockSpec((RB, D), lambda p, i: (i, 0)),
        out_shape=jax.ShapeDtypeStruct((NP, D), _f32),
        scratch_shapes=[pltpu.VMEM((8, D), _f32)],
    )(agg, sparts, b_gat, bn1_g, bn1_b)


def _mm_body(h, w, out):
    out[...] = jnp.dot(h[...], w[0], preferred_element_type=_f32)[None]


def _mm_call(h, wstack, nb, din):
    return pl.pallas_call(
        _mm_body,
        grid=(nb, NRB),
        in_specs=[
            pl.BlockSpec((RB, din), lambda j, r: (r, 0)),
            pl.BlockSpec((1, din, HID), lambda j, r: (j, 0, 0)),
        ],
        out_specs=pl.BlockSpec((1, RB, HID), lambda j, r: (j, r, 0)),
        out_shape=jax.ShapeDtypeStruct((nb, NP, HID), _f32),
    )(h, wstack)


def _m4_body(agg, hp, dinv, bc, g, b, out, acc):
    p = pl.program_id(0)
    i = pl.program_id(1)
    dv = dinv[:, :NE]
    a6 = agg[...].swapaxes(0, 1)
    at = jnp.concatenate([a6[:, :NE - 1],
                          (a6[:, NE - 1:NE] + a6[:, NE:NE + 1])], axis=1)
    ht = hp[...].swapaxes(0, 1)
    y3 = at * dv[:, :, None] + ht * (dv * dv)[:, :, None] + bc[...][None]
    y = y3.reshape(RB, NE * HID)

    @pl.when(p == 0)
    def _():
        @pl.when(i == 0)
        def _():
            acc[...] = jnp.zeros_like(acc)
        acc[0] = acc[0] + y.sum(0)
        acc[1] = acc[1] + (y * y).sum(0)

    @pl.when(p == 1)
    def _():
        mean = acc[0] / N
        var = acc[1] / N - mean * mean
        yn = (y - mean) * lax.rsqrt(var + 1e-5) * g[0] + b[0]
        out[...] = jnp.where(_rowmask(i), jnp.maximum(yn, 0.0), 0.0)


def _m4_call(agg, hp, dinv16, b_c, bn2_g, bn2_b):
    D = NE * HID
    return pl.pallas_call(
        _m4_body,
        grid=(2, NRB),
        in_specs=[
            pl.BlockSpec((NE + 1, RB, HID), lambda p, i: (0, i, 0)),
            pl.BlockSpec((NE, RB, HID), lambda p, i: (0, i, 0)),
            pl.BlockSpec((RB, HID), lambda p, i: (i, 0)),
            pl.BlockSpec((NE, HID), lambda p, i: (0, 0)),
            pl.BlockSpec((1, D), lambda p, i: (0, 0)),
            pl.BlockSpec((1, D), lambda p, i: (0, 0)),
        ],
        out_specs=pl.BlockSpec((RB, D), lambda p, i: (i, 0)),
        out_shape=jax.ShapeDtypeStruct((NP, D), _f32),
        scratch_shapes=[pltpu.VMEM((8, D), _f32)],
    )(agg, hp, dinv16, b_c, bn2_g, bn2_b)


def _m6_body(agg, hp, dinv, bml, g, b, omu, ols, acc):
    p = pl.program_id(0)
    i = pl.program_id(1)
    dv = dinv[:, :NE]
    dvv = jnp.concatenate([dv, dv], axis=1)
    at = agg[...].swapaxes(0, 1)
    ht = hp[...].swapaxes(0, 1)
    y3 = at * dvv[:, :, None] + ht * (dvv * dvv)[:, :, None] + bml[...][None]
    D = NE * HID
    ymu = y3[:, :NE].reshape(RB, D)
    yls = y3[:, NE:].reshape(RB, D)

    @pl.when(p == 0)
    def _():
        @pl.when(i == 0)
        def _():
            acc[...] = jnp.zeros_like(acc)
        acc[0] = acc[0] + ymu.sum(0)
        acc[1] = acc[1] + (ymu * ymu).sum(0)
        acc[2] = acc[2] + yls.sum(0)
        acc[3] = acc[3] + (yls * yls).sum(0)

    @pl.when(p == 1)
    def _():
        m = _rowmask(i)
        mu_m = acc[0] / N
        mu_v = acc[1] / N - mu_m * mu_m
        ls_m = acc[2] / N
        ls_v = acc[3] / N - ls_m * ls_m
        a = (ymu - mu_m) * lax.rsqrt(mu_v + 1e-5) * g[0] + b[0]
        c = (yls - ls_m) * lax.rsqrt(ls_v + 1e-5) * g[0] + b[0]
        omu[...] = jnp.where(m, jnp.maximum(a, 0.0), 0.0)
        ols[...] = jnp.where(m, jnp.maximum(c, 0.0), 0.0)


def _m6_call(agg, hp, dinv16, b_ml, bn2_g, bn2_b):
    D = NE * HID
    return pl.pallas_call(
        _m6_body,
        grid=(2, NRB),
        in_specs=[
            pl.BlockSpec((2 * NE, RB, HID), lambda p, i: (0, i, 0)),
            pl.BlockSpec((2 * NE, RB, HID), lambda p, i: (0, i, 0)),
            pl.BlockSpec((RB, HID), lambda p, i: (i, 0)),
            pl.BlockSpec((2 * NE, HID), lambda p, i: (0, 0)),
            pl.BlockSpec((1, D), lambda p, i: (0, 0)),
            pl.BlockSpec((1, D), lambda p, i: (0, 0)),
        ],
        out_specs=[
            pl.BlockSpec((RB, D), lambda p, i: (i, 0)),
            pl.BlockSpec((RB, D), lambda p, i: (i, 0)),
        ],
        out_shape=[
            jax.ShapeDtypeStruct((NP, D), _f32),
            jax.ShapeDtypeStruct((NP, D), _f32),
        ],
        scratch_shapes=[pltpu.VMEM((8, D), _f32)],
    )(agg, hp, dinv16, b_ml, bn2_g, bn2_b)


EB = 512
NEB = ET // EB


def _m8_body(r0, r1, b0, w1, b1, w2, b2, out):
    wb = w1[...]
    z0 = jnp.maximum(r0[...] + b0[0], 0.0).astype(jnp.bfloat16)
    z1 = jnp.maximum(r1[...] + b0[0], 0.0).astype(jnp.bfloat16)
    t0 = jnp.maximum(jnp.dot(z0, wb, preferred_element_type=_f32) + b1[0], 0.0)
    t1 = jnp.maximum(jnp.dot(z1, wb, preferred_element_type=_f32) + b1[0], 0.0)
    out[...] = jnp.dot(0.5 * (t0 + t1), w2[...], preferred_element_type=_f32) + b2[0]


def _m8_call(r0, r1, b_l0, W_l1, b_l1, W_l2p, b_l2p):
    es = pl.BlockSpec((EB, LH), lambda i: (i, 0))
    return pl.pallas_call(
        _m8_body,
        grid=(NEB,),
        in_specs=[
            es, es,
            pl.BlockSpec((1, LH), lambda i: (0, 0)),
            pl.BlockSpec((LH, LH), lambda i: (0, 0)),
            pl.BlockSpec((1, LH), lambda i: (0, 0)),
            pl.BlockSpec((LH, HID), lambda i: (0, 0)),
            pl.BlockSpec((1, HID), lambda i: (0, 0)),
        ],
        out_specs=pl.BlockSpec((EB, HID), lambda i: (i, 0)),
        out_shape=jax.ShapeDtypeStruct((ET, HID), _f32),
    )(r0, r1, b_l0, W_l1, b_l1, W_l2p, b_l2p)


def _pad_idx(n_extra):
    return (N + (jnp.arange(n_extra, dtype=_i32) % (NP - N))).astype(_i32)


def kernel(x, edge_index, edge_attr, edge_index_test, W_gat, att_src, att_dst,
           b_gat, bn1_g, bn1_b, bn2_g, bn2_b, W_c, b_c, W_mu, b_mu, W_ls, b_ls,
           W_l0, b_l0, W_l1, b_l1, W_l2, b_l2):
    xp = jnp.pad(x, ((0, NP - N), (0, 0)))
    loop = jnp.arange(N, dtype=_i32)
    srcg = jnp.concatenate([edge_index[0], loop, _pad_idx(EG - E - N)])
    dstg = jnp.concatenate([edge_index[1], loop, _pad_idx(EG - E - N)])
    srcw = jnp.concatenate([edge_index[0], _pad_idx(EW - E)])
    dstw = jnp.concatenate([edge_index[1], _pad_idx(EW - E)])
    st = jnp.concatenate([edge_index_test[0], _pad_idx(ET - E)])
    dt = jnp.concatenate([edge_index_test[1], _pad_idx(ET - E)])
    wflat = jnp.pad(edge_attr, ((0, EW - E), (0, L - NE))).reshape(EW * L)
    z128 = jnp.zeros((NP, HID), _f32)

    W_mlp = jnp.concatenate([W_mu, W_ls], axis=0)
    b_mlp = jnp.concatenate([b_mu, b_ls], axis=0)
    W0ab = W_l0.reshape(2, NE * HID, LH)
    W_l2p = jnp.pad(W_l2, ((0, 0), (0, HID - 4)))
    b_l2p = jnp.pad(b_l2, (0, HID - 4)).reshape(1, HID)
    b_gat2 = b_gat.reshape(1, -1)
    bn1_g2, bn1_b2 = bn1_g.reshape(1, -1), bn1_b.reshape(1, -1)
    bn2_g2, bn2_b2 = bn2_g.reshape(1, -1), bn2_b.reshape(1, -1)
    b_l0_2, b_l1_2 = b_l0.reshape(1, -1), b_l1.reshape(1, -1)

    xlbm, als128, ald128 = _m1_call(xp, W_gat, att_src, att_dst)
    e_sc = _k1a_call(srcg, dstg, als128, ald128)
    sparts = _k2a_call(dstg, e_sc, z128, EG)
    agg1 = _k3_call(srcg, dstg, e_sc, xlbm.reshape(H * NP, HID), z128,
                    nbp=H, ncol=16, etot=EG)
    h1 = _m2_call(agg1.reshape(H, NP, HID), sparts.reshape(NC, NP, HID),
                  b_gat2, bn1_g2, bn1_b2)

    degp = _k2a_call(dstw, wflat, z128, EW)
    dinv128 = _m2b_call(degp.reshape(NC, NP, HID))
    scale = _k2b_call(srcw, wflat, dinv128)

    hp_c = _mm_call(h1, W_c, NE, H * HID)
    agg2 = _k3_call(srcw, dstw, scale, hp_c.reshape(NE * NP, HID), z128,
                    nbp=NE, ncol=16, etot=EW)
    h2 = _m4_call(agg2.reshape(NE + 1, NP, HID), hp_c, dinv128, b_c,
                  bn2_g2, bn2_b2)

    hp_ml = _mm_call(h2, W_mlp, 2 * NE, NE * HID)
    agg3 = _k3_call(srcw, dstw, scale, hp_ml.reshape(2 * NE * NP, HID), z128,
                    nbp=2 * NE, ncol=NE, etot=EW)
    xmu_p, xls_p = _m6_call(agg3.reshape(2 * NE, NP, HID), hp_ml, dinv128,
                            b_mlp, bn2_g2, bn2_b2)

    pq = _mm_call2(xmu_p, W0ab)
    r0, r1 = _k4_call(st, dt, pq.reshape(2 * NP, LH))
    dfull = _m8_call(r0, r1, b_l0_2, W_l1.astype(jnp.bfloat16),
                     b_l1_2, W_l2p, b_l2p)

    return (xmu_p[:N], xls_p[:N], dfull[:E, :4])


def _mm2_body(h, w, out):
    out[...] = jnp.dot(h[...], w[0], preferred_element_type=_f32)[None]


def _mm_call2(h, wstack):
    return pl.pallas_call(
        _mm2_body,
        grid=(2, NRB),
        in_specs=[
            pl.BlockSpec((RB, NE * HID), lambda j, r: (r, 0)),
            pl.BlockSpec((1, NE * HID, LH), lambda j, r: (j, 0, 0)),
        ],
        out_specs=pl.BlockSpec((1, RB, LH), lambda j, r: (j, r, 0)),
        out_shape=jax.ShapeDtypeStruct((2, NP, LH), _f32),
    )(h, wstack)


# speedup vs baseline: 8.7656x; 1.0317x over previous
import functools

import jax
import jax.numpy as jnp
from jax import lax
from jax.experimental import pallas as pl
from jax.experimental.pallas import tpu as pltpu
from jax.experimental.pallas import tpu_sc as plsc

N = 10000
NP = 10240
IN = 256
HID = 128
H = 4
NE = 5
LH = 256
E = 160000

NC, NS, L = 2, 16, 16
NW = NC * NS
CH = 128

EG = 172032
EW = 163840
ET = 163840
RPT = NP // NS

_f32 = jnp.float32
_i32 = jnp.int32


def _mesh():
    return plsc.VectorSubcoreMesh(core_axis_name="c", subcore_axis_name="s",
                                  num_cores=NC, num_subcores=NS)


def _k1a_body(src_r, dst_r, als_r, ald_r, e_out, si, di, av, bv, e16, sem):
    cid = lax.axis_index("c")
    sid = lax.axis_index("s")
    w = sid * NC + cid
    nch = EG // (NW * CH)

    def chunk(ch, _):
        base = w * (EG // NW) + ch * CH
        pltpu.sync_copy(src_r.at[pl.ds(base, CH)], si)
        pltpu.sync_copy(dst_r.at[pl.ds(base, CH)], di)
        c0 = pltpu.async_copy(als_r.at[si], av, sem)
        c1 = pltpu.async_copy(ald_r.at[di], bv, sem)
        c0.wait()
        c1.wait()

        def row(r, c):
            v = av[r, pl.ds(0, L)] + bv[r, pl.ds(0, L)]
            v = jnp.where(v >= 0, v, 0.2 * v)
            e16[pl.ds(r * L, L)] = jnp.exp(v)
            return c
        lax.fori_loop(0, CH, row, 0)
        pltpu.sync_copy(e16, e_out.at[pl.ds(base * L, CH * L)])
        return _
    lax.fori_loop(0, nch, chunk, 0)


def _k1a_call(srcg, dstg, als128, ald128):
    f = pl.kernel(
        _k1a_body,
        out_type=jax.ShapeDtypeStruct((EG * L,), _f32),
        mesh=_mesh(),
        scratch_types=[
            pltpu.VMEM((CH,), _i32), pltpu.VMEM((CH,), _i32),
            pltpu.VMEM((CH, HID), _f32), pltpu.VMEM((CH, HID), _f32),
            pltpu.VMEM((CH * L,), _f32),
            pltpu.SemaphoreType.DMA,
        ],
    )
    return f(srcg, dstg, als128, ald128)


def _k2a_body(dst_r, w_r, z_r, d_out, di, wv, wv128, dacc, *, etot):
    cid = lax.axis_index("c")
    sid = lax.axis_index("s")
    w = sid * NC + cid
    rows0 = sid * RPT
    pltpu.sync_copy(z_r.at[pl.ds(rows0, RPT)], dacc.at[pl.ds(rows0, RPT)])

    def zrow(r, c):
        for j in range(1, HID // L):
            wv128[r, pl.ds(j * L, L)] = jnp.zeros((L,), _f32)
        return c
    lax.fori_loop(0, CH, zrow, 0)
    plsc.subcore_barrier()
    nch = etot // (NW * CH)

    def chunk(ch, _):
        base = w * (etot // NW) + ch * CH
        pltpu.sync_copy(dst_r.at[pl.ds(base, CH)], di)
        pltpu.sync_copy(w_r.at[pl.ds(base * L, CH * L)], wv)

        def row(r, c):
            wv128[r, pl.ds(0, L)] = wv[pl.ds(r * L, L)]
            return c
        lax.fori_loop(0, CH, row, 0)
        pltpu.sync_copy(wv128, dacc.at[di], add=True)
        return _
    lax.fori_loop(0, nch, chunk, 0)
    plsc.subcore_barrier()
    pltpu.sync_copy(dacc.at[pl.ds(rows0, RPT)],
                    d_out.at[pl.ds(cid * NP + rows0, RPT)])


def _k2a_call(dst, wflat, z128, etot):
    body = functools.partial(_k2a_body, etot=etot)
    f = pl.kernel(
        body,
        out_type=jax.ShapeDtypeStruct((NC * NP, HID), _f32),
        mesh=_mesh(),
        scratch_types=[
            pltpu.VMEM((CH,), _i32), pltpu.VMEM((CH * L,), _f32),
            pltpu.VMEM((CH, HID), _f32),
            pltpu.VMEM_SHARED((NP, HID), _f32),
        ],
    )
    return f(dst, wflat, z128)


def _k2b_body(src_r, w_r, dinv_r, s_out, si, wv, dv, sem):
    cid = lax.axis_index("c")
    sid = lax.axis_index("s")
    w = sid * NC + cid
    nch = EW // (NW * CH)

    def chunk(ch, _):
        base = w * (EW // NW) + ch * CH
        pltpu.sync_copy(src_r.at[pl.ds(base, CH)], si)
        pltpu.sync_copy(w_r.at[pl.ds(base * L, CH * L)], wv)
        pltpu.async_copy(dinv_r.at[si], dv, sem).wait()

        def row(r, c):
            wv[pl.ds(r * L, L)] = wv[pl.ds(r * L, L)] * dv[r, pl.ds(0, L)]
            return c
        lax.fori_loop(0, CH, row, 0)
        pltpu.sync_copy(wv, s_out.at[pl.ds(base * L, CH * L)])
        return _
    lax.fori_loop(0, nch, chunk, 0)


def _k2b_call(srcw, wflat, dinv128):
    f = pl.kernel(
        _k2b_body,
        out_type=jax.ShapeDtypeStruct((EW * L,), _f32),
        mesh=_mesh(),
        scratch_types=[
            pltpu.VMEM((CH,), _i32), pltpu.VMEM((CH * L,), _f32),
            pltpu.VMEM((CH, HID), _f32), pltpu.SemaphoreType.DMA,
        ],
    )
    return f(srcw, wflat, dinv128)


def _k3_body(src_r, dst_r, sc_r, tab_r, z_r, out_r,
             si0, si1, di0, di1, gi0, gi1, sv0, sv1, rw0, rw1, acc,
             sem0, sem1, semw0, semw1, *, nbp, ncol, etot):
    cid = lax.axis_index("c")
    sid = lax.axis_index("s")
    rows0 = sid * RPT
    ept = etot // NS
    nch = ept // CH
    sis, dis, gis, svs, rws = [si0, si1], [di0, di1], [gi0, gi1], \
        [sv0, sv1], [rw0, rw1]
    sems = [sem0, sem1]
    semws = [semw0, semw1]

    def stage(c, b, boff, wait_prev):
        if wait_prev is not False:
            @pl.when(wait_prev)
            def _w():
                pltpu.make_async_copy(rws[b], acc.at[dis[b]], semws[b]).wait()
        base = c
        pltpu.sync_copy(src_r.at[pl.ds(base, CH)], sis[b])
        pltpu.sync_copy(dst_r.at[pl.ds(base, CH)], dis[b])
        pltpu.sync_copy(sc_r.at[pl.ds(base * L, CH * L)], svs[b])

        def addi(r, cc):
            gis[b][pl.ds(r * L, L)] = sis[b][pl.ds(r * L, L)] + boff
            return cc
        lax.fori_loop(0, CH // L, addi, 0)
        pltpu.async_copy(tab_r.at[gis[b]], rws[b], sems[b])

    odd = nbp % 2 == 1
    trips = (nbp + NC - 1) // NC

    def block(k, carry):
        if odd:
            is_l = k == trips - 1
            b = jnp.where(is_l, nbp - 1, k * NC + cid)
            nch_eff = jnp.where(is_l, nch // 2, nch)
            ebase = sid * ept + jnp.where(is_l, cid * (ept // 2), 0)
            ooff = (b + jnp.where(is_l, cid, 0)) * NP
        else:
            b = k * NC + cid
            nch_eff = nch
            ebase = sid * ept
            ooff = b * NP
        boff = b * NP
        colv = jnp.full((L,), lax.rem(b, jnp.int32(ncol)), _i32)
        pltpu.sync_copy(z_r.at[pl.ds(rows0, RPT)], acc.at[pl.ds(rows0, RPT)])
        plsc.subcore_barrier()
        stage(ebase, 0, boff, False)

        def chunk2(cc, carry2):
            for bb in range(2):
                c = cc * 2 + bb

                @pl.when(c + 1 < nch_eff)
                def _stage_next():
                    stage(ebase + (c + 1) * CH, 1 - bb, boff, c + 1 >= 2)
                pltpu.make_async_copy(tab_r.at[gis[bb]], rws[bb],
                                      sems[bb]).wait()

                def edge(e4, c2):
                    for u in range(4):
                        e = e4 * 4 + u
                        v = svs[bb][pl.ds(e * L, L)]
                        sp = v[colv]
                        for j in range(HID // L):
                            rws[bb][e, pl.ds(j * L, L)] = \
                                rws[bb][e, pl.ds(j * L, L)] * sp
                    return c2
                lax.fori_loop(0, CH // 4, edge, 0)
                pltpu.async_copy(rws[bb], acc.at[dis[bb]], semws[bb],
                                 add=True)
            return carry2
        lax.fori_loop(0, nch_eff // 2, chunk2, 0)
        for b2 in range(2):
            pltpu.make_async_copy(rws[b2], acc.at[dis[b2]], semws[b2]).wait()
        plsc.subcore_barrier()
        pltpu.sync_copy(acc.at[pl.ds(rows0, RPT)],
                        out_r.at[pl.ds(ooff + rows0, RPT)])
        plsc.subcore_barrier()
        return carry
    lax.fori_loop(0, trips, block, 0)


def _k3_call(src, dst, scale, table, z128, *, nbp, ncol, etot):
    body = functools.partial(_k3_body, nbp=nbp, ncol=ncol, etot=etot)
    nbo = nbp + (nbp % 2)
    f = pl.kernel(
        body,
        out_type=jax.ShapeDtypeStruct((nbo * NP, HID), _f32),
        mesh=_mesh(),
        scratch_types=[
            pltpu.VMEM((CH,), _i32), pltpu.VMEM((CH,), _i32),
            pltpu.VMEM((CH,), _i32), pltpu.VMEM((CH,), _i32),
            pltpu.VMEM((CH,), _i32), pltpu.VMEM((CH,), _i32),
            pltpu.VMEM((CH * L,), _f32), pltpu.VMEM((CH * L,), _f32),
            pltpu.VMEM((CH, HID), _f32), pltpu.VMEM((CH, HID), _f32),
            pltpu.VMEM_SHARED((NP, HID), _f32),
            pltpu.SemaphoreType.DMA, pltpu.SemaphoreType.DMA,
            pltpu.SemaphoreType.DMA, pltpu.SemaphoreType.DMA,
        ],
    )
    return f(src, dst, scale, table, z128)


CH4 = 32


def _k4_body(st_r, dt_r, pq_r, r0_out, r1_out,
             si0, si1, di0, di1, gs0, gs1, gd0, gd1,
             av0, av1, bv0, bv1, cv0, cv1, dv0, dv1,
             semg0, semg1, semw0, semw1):
    cid = lax.axis_index("c")
    sid = lax.axis_index("s")
    w = sid * NC + cid
    nch = ET // (NW * CH4)
    sis, dis = [si0, si1], [di0, di1]
    gss, gds = [gs0, gs1], [gd0, gd1]
    avs, bvs, cvs, dvs = [av0, av1], [bv0, bv1], [cv0, cv1], [dv0, dv1]
    semgs, semws = [semg0, semg1], [semw0, semw1]

    def stage(ch, b, wait_prev):
        if wait_prev is not False:
            @pl.when(wait_prev)
            def _w():
                base_p = 0
                pltpu.make_async_copy(avs[b], r0_out.at[pl.ds(base_p, CH4)],
                                      semws[b]).wait()
                pltpu.make_async_copy(cvs[b], r1_out.at[pl.ds(base_p, CH4)],
                                      semws[b]).wait()
        base = w * (ET // NW) + ch * CH4
        pltpu.sync_copy(st_r.at[pl.ds(base, CH4)], sis[b])
        pltpu.sync_copy(dt_r.at[pl.ds(base, CH4)], dis[b])

        def addi(r, c):
            gss[b][pl.ds(r * L, L)] = sis[b][pl.ds(r * L, L)] + NP
            gds[b][pl.ds(r * L, L)] = dis[b][pl.ds(r * L, L)] + NP
            return c
        lax.fori_loop(0, CH4 // L, addi, 0)
        pltpu.async_copy(pq_r.at[sis[b]], avs[b], semgs[b])
        pltpu.async_copy(pq_r.at[gds[b]], bvs[b], semgs[b])
        pltpu.async_copy(pq_r.at[dis[b]], cvs[b], semgs[b])
        pltpu.async_copy(pq_r.at[gss[b]], dvs[b], semgs[b])

    stage(0, 0, False)

    def chunk2(cc, carry):
        for bb in range(2):
            c = cc * 2 + bb
            base = w * (ET // NW) + c * CH4

            @pl.when(c + 1 < nch)
            def _stage_next():
                stage(c + 1, 1 - bb, c + 1 >= 2)
            for buf in (avs, bvs, cvs, dvs):
                pltpu.make_async_copy(pq_r.at[sis[bb]], buf[bb],
                                      semgs[bb]).wait()

            def row(r, c2):
                for j in range(LH // L):
                    avs[bb][r, pl.ds(j * L, L)] = \
                        avs[bb][r, pl.ds(j * L, L)] + bvs[bb][r, pl.ds(j * L, L)]
                    cvs[bb][r, pl.ds(j * L, L)] = \
                        cvs[bb][r, pl.ds(j * L, L)] + dvs[bb][r, pl.ds(j * L, L)]
                return c2
            lax.fori_loop(0, CH4, row, 0)
            pltpu.async_copy(avs[bb], r0_out.at[pl.ds(base, CH4)], semws[bb])
            pltpu.async_copy(cvs[bb], r1_out.at[pl.ds(base, CH4)], semws[bb])
        return carry
    lax.fori_loop(0, nch // 2, chunk2, 0)
    for b2 in range(2):
        pltpu.make_async_copy(avs[b2], r0_out.at[pl.ds(0, CH4)],
                              semws[b2]).wait()
        pltpu.make_async_copy(cvs[b2], r1_out.at[pl.ds(0, CH4)],
                              semws[b2]).wait()


def _k4_call(st, dt, pq):
    o = jax.ShapeDtypeStruct((ET, LH), _f32)
    f = pl.kernel(
        _k4_body,
        out_type=(o, o),
        mesh=_mesh(),
        scratch_types=(
            [pltpu.VMEM((CH4,), _i32)] * 8
            + [pltpu.VMEM((CH4, LH), _f32)] * 8
            + [pltpu.SemaphoreType.DMA] * 4
        ),
    )
    return f(st, dt, pq)


RB = 256
NRB = NP // RB


def _m1_body(xp, wg, ats, atd, xlbm, als128, ald128):
    a = jnp.dot(xp[...], wg[...], preferred_element_type=_f32)
    a3 = a.reshape(RB, H, HID)
    xlbm[...] = a3.swapaxes(0, 1)
    als = (a3 * ats[...][None]).sum(-1)
    ald = (a3 * atd[...][None]).sum(-1)
    z = jnp.zeros((RB, HID - H), _f32)
    als128[...] = jnp.concatenate([als, z], axis=1)
    ald128[...] = jnp.concatenate([ald, z], axis=1)


def _m1_call(xp, W_gat, att_src, att_dst):
    return pl.pallas_call(
        _m1_body,
        grid=(NRB,),
        in_specs=[
            pl.BlockSpec((RB, IN), lambda i: (i, 0)),
            pl.BlockSpec((IN, H * HID), lambda i: (0, 0)),
            pl.BlockSpec((H, HID), lambda i: (0, 0)),
            pl.BlockSpec((H, HID), lambda i: (0, 0)),
        ],
        out_specs=[
            pl.BlockSpec((H, RB, HID), lambda i: (0, i, 0)),
            pl.BlockSpec((RB, HID), lambda i: (i, 0)),
            pl.BlockSpec((RB, HID), lambda i: (i, 0)),
        ],
        out_shape=[
            jax.ShapeDtypeStruct((H, NP, HID), _f32),
            jax.ShapeDtypeStruct((NP, HID), _f32),
            jax.ShapeDtypeStruct((NP, HID), _f32),
        ],
    )(xp, W_gat, att_src, att_dst)


def _m2b_body(degp, dinv128):
    d = degp[0] + degp[1] + 1.0
    dinv128[...] = lax.rsqrt(d)


def _m2b_call(degp):
    return pl.pallas_call(
        _m2b_body,
        grid=(NRB,),
        in_specs=[pl.BlockSpec((NC, RB, HID), lambda i: (0, i, 0))],
        out_specs=pl.BlockSpec((RB, HID), lambda i: (i, 0)),
        out_shape=jax.ShapeDtypeStruct((NP, HID), _f32),
    )(degp)


def _rowmask(i):
    rows = i * RB + lax.broadcasted_iota(_i32, (RB, 1), 0)
    return rows < N


def _m2_body(agg, sp, bg, g, b, out, acc):
    p = pl.program_id(0)
    i = pl.program_id(1)
    s = sp[0] + sp[1]
    s4 = s[:, :H]
    at = agg[...].swapaxes(0, 1)
    y3 = at / (s4[:, :, None] + 1e-16)
    y = y3.reshape(RB, H * HID) + bg[0]

    @pl.when(p == 0)
    def _():
        @pl.when(i == 0)
        def _():
            acc[...] = jnp.zeros_like(acc)
        acc[0] = acc[0] + y.sum(0)
        acc[1] = acc[1] + (y * y).sum(0)

    @pl.when(p == 1)
    def _():
        mean = acc[0] / N
        var = acc[1] / N - mean * mean
        yn = (y - mean) * lax.rsqrt(var + 1e-5) * g[0] + b[0]
        out[...] = jnp.where(_rowmask(i), jnp.maximum(yn, 0.0), 0.0)


def _m2_call(agg, sparts, b_gat, bn1_g, bn1_b):
    D = H * HID
    return pl.pallas_call(
        _m2_body,
        grid=(2, NRB),
        in_specs=[
            pl.BlockSpec((H, RB, HID), lambda p, i: (0, i, 0)),
            pl.BlockSpec((NC, RB, HID), lambda p, i: (0, i, 0)),
            pl.BlockSpec((1, D), lambda p, i: (0, 0)),
            pl.BlockSpec((1, D), lambda p, i: (0, 0)),
            pl.BlockSpec((1, D), lambda p, i: (0, 0)),
        ],
        out_specs=pl.BlockSpec((RB, D), lambda p, i: (i, 0)),
        out_shape=jax.ShapeDtypeStruct((NP, D), _f32),
        scratch_shapes=[pltpu.VMEM((8, D), _f32)],
    )(agg, sparts, b_gat, bn1_g, bn1_b)


def _mm_body(h, w, out):
    out[...] = jnp.dot(h[...], w[0], preferred_element_type=_f32)[None]


def _mm_call(h, wstack, nb, din):
    return pl.pallas_call(
        _mm_body,
        grid=(nb, NRB),
        in_specs=[
            pl.BlockSpec((RB, din), lambda j, r: (r, 0)),
            pl.BlockSpec((1, din, HID), lambda j, r: (j, 0, 0)),
        ],
        out_specs=pl.BlockSpec((1, RB, HID), lambda j, r: (j, r, 0)),
        out_shape=jax.ShapeDtypeStruct((nb, NP, HID), _f32),
    )(h, wstack)


def _m4_body(agg, hp, dinv, bc, g, b, out, acc):
    p = pl.program_id(0)
    i = pl.program_id(1)
    dv = dinv[:, :NE]
    a6 = agg[...].swapaxes(0, 1)
    at = jnp.concatenate([a6[:, :NE - 1],
                          (a6[:, NE - 1:NE] + a6[:, NE:NE + 1])], axis=1)
    ht = hp[...].swapaxes(0, 1)
    y3 = at * dv[:, :, None] + ht * (dv * dv)[:, :, None] + bc[...][None]
    y = y3.reshape(RB, NE * HID)

    @pl.when(p == 0)
    def _():
        @pl.when(i == 0)
        def _():
            acc[...] = jnp.zeros_like(acc)
        acc[0] = acc[0] + y.sum(0)
        acc[1] = acc[1] + (y * y).sum(0)

    @pl.when(p == 1)
    def _():
        mean = acc[0] / N
        var = acc[1] / N - mean * mean
        yn = (y - mean) * lax.rsqrt(var + 1e-5) * g[0] + b[0]
        out[...] = jnp.where(_rowmask(i), jnp.maximum(yn, 0.0), 0.0)


def _m4_call(agg, hp, dinv16, b_c, bn2_g, bn2_b):
    D = NE * HID
    return pl.pallas_call(
        _m4_body,
        grid=(2, NRB),
        in_specs=[
            pl.BlockSpec((NE + 1, RB, HID), lambda p, i: (0, i, 0)),
            pl.BlockSpec((NE, RB, HID), lambda p, i: (0, i, 0)),
            pl.BlockSpec((RB, HID), lambda p, i: (i, 0)),
            pl.BlockSpec((NE, HID), lambda p, i: (0, 0)),
            pl.BlockSpec((1, D), lambda p, i: (0, 0)),
            pl.BlockSpec((1, D), lambda p, i: (0, 0)),
        ],
        out_specs=pl.BlockSpec((RB, D), lambda p, i: (i, 0)),
        out_shape=jax.ShapeDtypeStruct((NP, D), _f32),
        scratch_shapes=[pltpu.VMEM((8, D), _f32)],
    )(agg, hp, dinv16, b_c, bn2_g, bn2_b)


def _m6_body(agg, hp, dinv, bml, g, b, omu, ols, acc):
    p = pl.program_id(0)
    i = pl.program_id(1)
    dv = dinv[:, :NE]
    dvv = jnp.concatenate([dv, dv], axis=1)
    at = agg[...].swapaxes(0, 1)
    ht = hp[...].swapaxes(0, 1)
    y3 = at * dvv[:, :, None] + ht * (dvv * dvv)[:, :, None] + bml[...][None]
    D = NE * HID
    ymu = y3[:, :NE].reshape(RB, D)
    yls = y3[:, NE:].reshape(RB, D)

    @pl.when(p == 0)
    def _():
        @pl.when(i == 0)
        def _():
            acc[...] = jnp.zeros_like(acc)
        acc[0] = acc[0] + ymu.sum(0)
        acc[1] = acc[1] + (ymu * ymu).sum(0)
        acc[2] = acc[2] + yls.sum(0)
        acc[3] = acc[3] + (yls * yls).sum(0)

    @pl.when(p == 1)
    def _():
        m = _rowmask(i)
        mu_m = acc[0] / N
        mu_v = acc[1] / N - mu_m * mu_m
        ls_m = acc[2] / N
        ls_v = acc[3] / N - ls_m * ls_m
        a = (ymu - mu_m) * lax.rsqrt(mu_v + 1e-5) * g[0] + b[0]
        c = (yls - ls_m) * lax.rsqrt(ls_v + 1e-5) * g[0] + b[0]
        omu[...] = jnp.where(m, jnp.maximum(a, 0.0), 0.0)
        ols[...] = jnp.where(m, jnp.maximum(c, 0.0), 0.0)


def _m6_call(agg, hp, dinv16, b_ml, bn2_g, bn2_b):
    D = NE * HID
    return pl.pallas_call(
        _m6_body,
        grid=(2, NRB),
        in_specs=[
            pl.BlockSpec((2 * NE, RB, HID), lambda p, i: (0, i, 0)),
            pl.BlockSpec((2 * NE, RB, HID), lambda p, i: (0, i, 0)),
            pl.BlockSpec((RB, HID), lambda p, i: (i, 0)),
            pl.BlockSpec((2 * NE, HID), lambda p, i: (0, 0)),
            pl.BlockSpec((1, D), lambda p, i: (0, 0)),
            pl.BlockSpec((1, D), lambda p, i: (0, 0)),
        ],
        out_specs=[
            pl.BlockSpec((RB, D), lambda p, i: (i, 0)),
            pl.BlockSpec((RB, D), lambda p, i: (i, 0)),
        ],
        out_shape=[
            jax.ShapeDtypeStruct((NP, D), _f32),
            jax.ShapeDtypeStruct((NP, D), _f32),
        ],
        scratch_shapes=[pltpu.VMEM((8, D), _f32)],
    )(agg, hp, dinv16, b_ml, bn2_g, bn2_b)


EB = 512
NEB = ET // EB


def _m8_body(r0, r1, b0, w1, b1, w2, b2, out):
    wb = w1[...]
    z0 = jnp.maximum(r0[...] + b0[0], 0.0).astype(jnp.bfloat16)
    z1 = jnp.maximum(r1[...] + b0[0], 0.0).astype(jnp.bfloat16)
    t0 = jnp.maximum(jnp.dot(z0, wb, preferred_element_type=_f32) + b1[0], 0.0)
    t1 = jnp.maximum(jnp.dot(z1, wb, preferred_element_type=_f32) + b1[0], 0.0)
    out[...] = jnp.dot(0.5 * (t0 + t1), w2[...], preferred_element_type=_f32) + b2[0]


def _m8_call(r0, r1, b_l0, W_l1, b_l1, W_l2p, b_l2p):
    es = pl.BlockSpec((EB, LH), lambda i: (i, 0))
    return pl.pallas_call(
        _m8_body,
        grid=(NEB,),
        in_specs=[
            es, es,
            pl.BlockSpec((1, LH), lambda i: (0, 0)),
            pl.BlockSpec((LH, LH), lambda i: (0, 0)),
            pl.BlockSpec((1, LH), lambda i: (0, 0)),
            pl.BlockSpec((LH, HID), lambda i: (0, 0)),
            pl.BlockSpec((1, HID), lambda i: (0, 0)),
        ],
        out_specs=pl.BlockSpec((EB, HID), lambda i: (i, 0)),
        out_shape=jax.ShapeDtypeStruct((ET, HID), _f32),
    )(r0, r1, b_l0, W_l1, b_l1, W_l2p, b_l2p)


def _pad_idx(n_extra):
    return (N + (jnp.arange(n_extra, dtype=_i32) % (NP - N))).astype(_i32)


def kernel(x, edge_index, edge_attr, edge_index_test, W_gat, att_src, att_dst,
           b_gat, bn1_g, bn1_b, bn2_g, bn2_b, W_c, b_c, W_mu, b_mu, W_ls, b_ls,
           W_l0, b_l0, W_l1, b_l1, W_l2, b_l2):
    xp = jnp.pad(x, ((0, NP - N), (0, 0)))
    loop = jnp.arange(N, dtype=_i32)
    srcg = jnp.concatenate([edge_index[0], loop, _pad_idx(EG - E - N)])
    dstg = jnp.concatenate([edge_index[1], loop, _pad_idx(EG - E - N)])
    srcw = jnp.concatenate([edge_index[0], _pad_idx(EW - E)])
    dstw = jnp.concatenate([edge_index[1], _pad_idx(EW - E)])
    st = jnp.concatenate([edge_index_test[0], _pad_idx(ET - E)])
    dt = jnp.concatenate([edge_index_test[1], _pad_idx(ET - E)])
    wflat = jnp.pad(edge_attr, ((0, EW - E), (0, L - NE))).reshape(EW * L)
    z128 = jnp.zeros((NP, HID), _f32)

    W_mlp = jnp.concatenate([W_mu, W_ls], axis=0)
    b_mlp = jnp.concatenate([b_mu, b_ls], axis=0)
    W0ab = W_l0.reshape(2, NE * HID, LH)
    W_l2p = jnp.pad(W_l2, ((0, 0), (0, HID - 4)))
    b_l2p = jnp.pad(b_l2, (0, HID - 4)).reshape(1, HID)
    b_gat2 = b_gat.reshape(1, -1)
    bn1_g2, bn1_b2 = bn1_g.reshape(1, -1), bn1_b.reshape(1, -1)
    bn2_g2, bn2_b2 = bn2_g.reshape(1, -1), bn2_b.reshape(1, -1)
    b_l0_2, b_l1_2 = b_l0.reshape(1, -1), b_l1.reshape(1, -1)

    xlbm, als128, ald128 = _m1_call(xp, W_gat, att_src, att_dst)
    e_sc = _k1a_call(srcg, dstg, als128, ald128)
    sparts = _k2a_call(dstg, e_sc, z128, EG)
    agg1 = _k3_call(srcg, dstg, e_sc, xlbm.reshape(H * NP, HID), z128,
                    nbp=H, ncol=16, etot=EG)
    h1 = _m2_call(agg1.reshape(H, NP, HID), sparts.reshape(NC, NP, HID),
                  b_gat2, bn1_g2, bn1_b2)

    degp = _k2a_call(dstw, wflat, z128, EW)
    dinv128 = _m2b_call(degp.reshape(NC, NP, HID))
    scale = _k2b_call(srcw, wflat, dinv128)

    hp_c = _mm_call(h1, W_c, NE, H * HID)
    agg2 = _k3_call(srcw, dstw, scale, hp_c.reshape(NE * NP, HID), z128,
                    nbp=NE, ncol=16, etot=EW)
    h2 = _m4_call(agg2.reshape(NE + 1, NP, HID), hp_c, dinv128, b_c,
                  bn2_g2, bn2_b2)

    hp_ml = _mm_call(h2, W_mlp, 2 * NE, NE * HID)
    agg3 = _k3_call(srcw, dstw, scale, hp_ml.reshape(2 * NE * NP, HID), z128,
                    nbp=2 * NE, ncol=NE, etot=EW)
    xmu_p, xls_p = _m6_call(agg3.reshape(2 * NE, NP, HID), hp_ml, dinv128,
                            b_mlp, bn2_g2, bn2_b2)

    pq = _mm_call2(xmu_p, W0ab)
    r0, r1 = _k4_call(st, dt, pq.reshape(2 * NP, LH))
    dfull = _m8_call(r0, r1, b_l0_2, W_l1.astype(jnp.bfloat16),
                     b_l1_2, W_l2p, b_l2p)

    return (xmu_p[:N], xls_p[:N], dfull[:E, :4])


def _mm2_body(h, w, out):
    out[...] = jnp.dot(h[...], w[0], preferred_element_type=_f32)[None]


def _mm_call2(h, wstack):
    return pl.pallas_call(
        _mm2_body,
        grid=(2, NRB),
        in_specs=[
            pl.BlockSpec((RB, NE * HID), lambda j, r: (r, 0)),
            pl.BlockSpec((1, NE * HID, LH), lambda j, r: (j, 0, 0)),
        ],
        out_specs=pl.BlockSpec((1, RB, LH), lambda j, r: (j, r, 0)),
        out_shape=jax.ShapeDtypeStruct((2, NP, LH), _f32),
    )(h, wstack)


# speedup vs baseline: 8.8532x; 1.0100x over previous
import functools

import jax
import jax.numpy as jnp
from jax import lax
from jax.experimental import pallas as pl
from jax.experimental.pallas import tpu as pltpu
from jax.experimental.pallas import tpu_sc as plsc

N = 10000
NP = 10240
IN = 256
HID = 128
H = 4
NE = 5
LH = 256
E = 160000

NC, NS, L = 2, 16, 16
NW = NC * NS
CH = 128

EG = 172032
EW = 163840
ET = 163840
RPT = NP // NS

_f32 = jnp.float32
_i32 = jnp.int32


def _mesh():
    return plsc.VectorSubcoreMesh(core_axis_name="c", subcore_axis_name="s",
                                  num_cores=NC, num_subcores=NS)


def _k1a_body(src_r, dst_r, als_r, ald_r, e_out,
              si0, si1, di0, di1, av0, av1, bv0, bv1, e0, e1,
              semg0, semg1, semw0, semw1):
    cid = lax.axis_index("c")
    sid = lax.axis_index("s")
    w = sid * NC + cid
    nch = EG // (NW * CH)
    sis, dis = [si0, si1], [di0, di1]
    avs, bvs, e16s = [av0, av1], [bv0, bv1], [e0, e1]
    semgs, semws = [semg0, semg1], [semw0, semw1]

    def stage(ch, b, wait_prev):
        if wait_prev is not False:
            @pl.when(wait_prev)
            def _w():
                pltpu.make_async_copy(e16s[b], e_out.at[pl.ds(0, CH * L)],
                                      semws[b]).wait()
        base = w * (EG // NW) + ch * CH
        pltpu.sync_copy(src_r.at[pl.ds(base, CH)], sis[b])
        pltpu.sync_copy(dst_r.at[pl.ds(base, CH)], dis[b])
        pltpu.async_copy(als_r.at[sis[b]], avs[b], semgs[b])
        pltpu.async_copy(ald_r.at[dis[b]], bvs[b], semgs[b])

    stage(0, 0, False)

    def chunk2(cc, carry):
        for bb in range(2):
            c = cc * 2 + bb
            base = w * (EG // NW) + c * CH

            @pl.when(c + 1 < nch)
            def _stage_next():
                stage(c + 1, 1 - bb, c + 1 >= 2)
            pltpu.make_async_copy(als_r.at[sis[bb]], avs[bb], semgs[bb]).wait()
            pltpu.make_async_copy(ald_r.at[dis[bb]], bvs[bb], semgs[bb]).wait()

            def row(r, c2):
                v = avs[bb][r, pl.ds(0, L)] + bvs[bb][r, pl.ds(0, L)]
                v = jnp.where(v >= 0, v, 0.2 * v)
                e16s[bb][pl.ds(r * L, L)] = jnp.exp(v)
                return c2
            lax.fori_loop(0, CH, row, 0)
            pltpu.async_copy(e16s[bb], e_out.at[pl.ds(base * L, CH * L)],
                             semws[bb])
        return carry
    lax.fori_loop(0, nch // 2, chunk2, 0)
    for b2 in range(2):
        pltpu.make_async_copy(e16s[b2], e_out.at[pl.ds(0, CH * L)],
                              semws[b2]).wait()


def _k1a_call(srcg, dstg, als128, ald128):
    f = pl.kernel(
        _k1a_body,
        out_type=jax.ShapeDtypeStruct((EG * L,), _f32),
        mesh=_mesh(),
        scratch_types=(
            [pltpu.VMEM((CH,), _i32)] * 4
            + [pltpu.VMEM((CH, HID), _f32)] * 4
            + [pltpu.VMEM((CH * L,), _f32)] * 2
            + [pltpu.SemaphoreType.DMA] * 4
        ),
    )
    return f(srcg, dstg, als128, ald128)


def _k2a_body(dst_r, w_r, z_r, d_out, di, wv, wv128, dacc, *, etot):
    cid = lax.axis_index("c")
    sid = lax.axis_index("s")
    w = sid * NC + cid
    rows0 = sid * RPT
    pltpu.sync_copy(z_r.at[pl.ds(rows0, RPT)], dacc.at[pl.ds(rows0, RPT)])

    def zrow(r, c):
        for j in range(1, HID // L):
            wv128[r, pl.ds(j * L, L)] = jnp.zeros((L,), _f32)
        return c
    lax.fori_loop(0, CH, zrow, 0)
    plsc.subcore_barrier()
    nch = etot // (NW * CH)

    def chunk(ch, _):
        base = w * (etot // NW) + ch * CH
        pltpu.sync_copy(dst_r.at[pl.ds(base, CH)], di)
        pltpu.sync_copy(w_r.at[pl.ds(base * L, CH * L)], wv)

        def row(r, c):
            wv128[r, pl.ds(0, L)] = wv[pl.ds(r * L, L)]
            return c
        lax.fori_loop(0, CH, row, 0)
        pltpu.sync_copy(wv128, dacc.at[di], add=True)
        return _
    lax.fori_loop(0, nch, chunk, 0)
    plsc.subcore_barrier()
    pltpu.sync_copy(dacc.at[pl.ds(rows0, RPT)],
                    d_out.at[pl.ds(cid * NP + rows0, RPT)])


def _k2a_call(dst, wflat, z128, etot):
    body = functools.partial(_k2a_body, etot=etot)
    f = pl.kernel(
        body,
        out_type=jax.ShapeDtypeStruct((NC * NP, HID), _f32),
        mesh=_mesh(),
        scratch_types=[
            pltpu.VMEM((CH,), _i32), pltpu.VMEM((CH * L,), _f32),
            pltpu.VMEM((CH, HID), _f32),
            pltpu.VMEM_SHARED((NP, HID), _f32),
        ],
    )
    return f(dst, wflat, z128)


def _k2b_body(src_r, w_r, dinv_r, s_out, si, wv, dv, sem):
    cid = lax.axis_index("c")
    sid = lax.axis_index("s")
    w = sid * NC + cid
    nch = EW // (NW * CH)

    def chunk(ch, _):
        base = w * (EW // NW) + ch * CH
        pltpu.sync_copy(src_r.at[pl.ds(base, CH)], si)
        pltpu.sync_copy(w_r.at[pl.ds(base * L, CH * L)], wv)
        pltpu.async_copy(dinv_r.at[si], dv, sem).wait()

        def row(r, c):
            wv[pl.ds(r * L, L)] = wv[pl.ds(r * L, L)] * dv[r, pl.ds(0, L)]
            return c
        lax.fori_loop(0, CH, row, 0)
        pltpu.sync_copy(wv, s_out.at[pl.ds(base * L, CH * L)])
        return _
    lax.fori_loop(0, nch, chunk, 0)


def _k2b_call(srcw, wflat, dinv128):
    f = pl.kernel(
        _k2b_body,
        out_type=jax.ShapeDtypeStruct((EW * L,), _f32),
        mesh=_mesh(),
        scratch_types=[
            pltpu.VMEM((CH,), _i32), pltpu.VMEM((CH * L,), _f32),
            pltpu.VMEM((CH, HID), _f32), pltpu.SemaphoreType.DMA,
        ],
    )
    return f(srcw, wflat, dinv128)


def _k3_body(src_r, dst_r, sc_r, tab_r, z_r, out_r,
             si0, si1, di0, di1, gi0, gi1, sv0, sv1, rw0, rw1, acc,
             sem0, sem1, semw0, semw1, *, nbp, ncol, etot):
    cid = lax.axis_index("c")
    sid = lax.axis_index("s")
    rows0 = sid * RPT
    ept = etot // NS
    nch = ept // CH
    sis, dis, gis, svs, rws = [si0, si1], [di0, di1], [gi0, gi1], \
        [sv0, sv1], [rw0, rw1]
    sems = [sem0, sem1]
    semws = [semw0, semw1]

    def stage(c, b, boff, wait_prev):
        if wait_prev is not False:
            @pl.when(wait_prev)
            def _w():
                pltpu.make_async_copy(rws[b], acc.at[dis[b]], semws[b]).wait()
        base = c
        pltpu.sync_copy(src_r.at[pl.ds(base, CH)], sis[b])
        pltpu.sync_copy(dst_r.at[pl.ds(base, CH)], dis[b])
        pltpu.sync_copy(sc_r.at[pl.ds(base * L, CH * L)], svs[b])

        def addi(r, cc):
            gis[b][pl.ds(r * L, L)] = sis[b][pl.ds(r * L, L)] + boff
            return cc
        lax.fori_loop(0, CH // L, addi, 0)
        pltpu.async_copy(tab_r.at[gis[b]], rws[b], sems[b])

    odd = nbp % 2 == 1
    trips = (nbp + NC - 1) // NC

    def block(k, carry):
        if odd:
            is_l = k == trips - 1
            b = jnp.where(is_l, nbp - 1, k * NC + cid)
            nch_eff = jnp.where(is_l, nch // 2, nch)
            ebase = sid * ept + jnp.where(is_l, cid * (ept // 2), 0)
            ooff = (b + jnp.where(is_l, cid, 0)) * NP
        else:
            b = k * NC + cid
            nch_eff = nch
            ebase = sid * ept
            ooff = b * NP
        boff = b * NP
        colv = jnp.full((L,), lax.rem(b, jnp.int32(ncol)), _i32)
        pltpu.sync_copy(z_r.at[pl.ds(rows0, RPT)], acc.at[pl.ds(rows0, RPT)])
        plsc.subcore_barrier()
        stage(ebase, 0, boff, False)

        def chunk2(cc, carry2):
            for bb in range(2):
                c = cc * 2 + bb

                @pl.when(c + 1 < nch_eff)
                def _stage_next():
                    stage(ebase + (c + 1) * CH, 1 - bb, boff, c + 1 >= 2)
                pltpu.make_async_copy(tab_r.at[gis[bb]], rws[bb],
                                      sems[bb]).wait()

                def edge(e4, c2):
                    for u in range(4):
                        e = e4 * 4 + u
                        v = svs[bb][pl.ds(e * L, L)]
                        sp = v[colv]
                        for j in range(HID // L):
                            rws[bb][e, pl.ds(j * L, L)] = \
                                rws[bb][e, pl.ds(j * L, L)] * sp
                    return c2
                lax.fori_loop(0, CH // 4, edge, 0)
                pltpu.async_copy(rws[bb], acc.at[dis[bb]], semws[bb],
                                 add=True)
            return carry2
        lax.fori_loop(0, nch_eff // 2, chunk2, 0)
        for b2 in range(2):
            pltpu.make_async_copy(rws[b2], acc.at[dis[b2]], semws[b2]).wait()
        plsc.subcore_barrier()
        pltpu.sync_copy(acc.at[pl.ds(rows0, RPT)],
                        out_r.at[pl.ds(ooff + rows0, RPT)])
        plsc.subcore_barrier()
        return carry
    lax.fori_loop(0, trips, block, 0)


def _k3_call(src, dst, scale, table, z128, *, nbp, ncol, etot):
    body = functools.partial(_k3_body, nbp=nbp, ncol=ncol, etot=etot)
    nbo = nbp + (nbp % 2)
    f = pl.kernel(
        body,
        out_type=jax.ShapeDtypeStruct((nbo * NP, HID), _f32),
        mesh=_mesh(),
        scratch_types=[
            pltpu.VMEM((CH,), _i32), pltpu.VMEM((CH,), _i32),
            pltpu.VMEM((CH,), _i32), pltpu.VMEM((CH,), _i32),
            pltpu.VMEM((CH,), _i32), pltpu.VMEM((CH,), _i32),
            pltpu.VMEM((CH * L,), _f32), pltpu.VMEM((CH * L,), _f32),
            pltpu.VMEM((CH, HID), _f32), pltpu.VMEM((CH, HID), _f32),
            pltpu.VMEM_SHARED((NP, HID), _f32),
            pltpu.SemaphoreType.DMA, pltpu.SemaphoreType.DMA,
            pltpu.SemaphoreType.DMA, pltpu.SemaphoreType.DMA,
        ],
    )
    return f(src, dst, scale, table, z128)


CH4 = 32


def _k4_body(st_r, dt_r, pq_r, r0_out, r1_out,
             si0, si1, di0, di1, gs0, gs1, gd0, gd1,
             av0, av1, bv0, bv1, cv0, cv1, dv0, dv1,
             semg0, semg1, semw0, semw1):
    cid = lax.axis_index("c")
    sid = lax.axis_index("s")
    w = sid * NC + cid
    nch = ET // (NW * CH4)
    sis, dis = [si0, si1], [di0, di1]
    gss, gds = [gs0, gs1], [gd0, gd1]
    avs, bvs, cvs, dvs = [av0, av1], [bv0, bv1], [cv0, cv1], [dv0, dv1]
    semgs, semws = [semg0, semg1], [semw0, semw1]

    def stage(ch, b, wait_prev):
        if wait_prev is not False:
            @pl.when(wait_prev)
            def _w():
                base_p = 0
                pltpu.make_async_copy(avs[b], r0_out.at[pl.ds(base_p, CH4)],
                                      semws[b]).wait()
                pltpu.make_async_copy(cvs[b], r1_out.at[pl.ds(base_p, CH4)],
                                      semws[b]).wait()
        base = w * (ET // NW) + ch * CH4
        pltpu.sync_copy(st_r.at[pl.ds(base, CH4)], sis[b])
        pltpu.sync_copy(dt_r.at[pl.ds(base, CH4)], dis[b])

        def addi(r, c):
            gss[b][pl.ds(r * L, L)] = sis[b][pl.ds(r * L, L)] + NP
            gds[b][pl.ds(r * L, L)] = dis[b][pl.ds(r * L, L)] + NP
            return c
        lax.fori_loop(0, CH4 // L, addi, 0)
        pltpu.async_copy(pq_r.at[sis[b]], avs[b], semgs[b])
        pltpu.async_copy(pq_r.at[gds[b]], bvs[b], semgs[b])
        pltpu.async_copy(pq_r.at[dis[b]], cvs[b], semgs[b])
        pltpu.async_copy(pq_r.at[gss[b]], dvs[b], semgs[b])

    stage(0, 0, False)

    def chunk2(cc, carry):
        for bb in range(2):
            c = cc * 2 + bb
            base = w * (ET // NW) + c * CH4

            @pl.when(c + 1 < nch)
            def _stage_next():
                stage(c + 1, 1 - bb, c + 1 >= 2)
            for buf in (avs, bvs, cvs, dvs):
                pltpu.make_async_copy(pq_r.at[sis[bb]], buf[bb],
                                      semgs[bb]).wait()

            def row(r, c2):
                for j in range(LH // L):
                    avs[bb][r, pl.ds(j * L, L)] = \
                        avs[bb][r, pl.ds(j * L, L)] + bvs[bb][r, pl.ds(j * L, L)]
                    cvs[bb][r, pl.ds(j * L, L)] = \
                        cvs[bb][r, pl.ds(j * L, L)] + dvs[bb][r, pl.ds(j * L, L)]
                return c2
            lax.fori_loop(0, CH4, row, 0)
            pltpu.async_copy(avs[bb], r0_out.at[pl.ds(base, CH4)], semws[bb])
            pltpu.async_copy(cvs[bb], r1_out.at[pl.ds(base, CH4)], semws[bb])
        return carry
    lax.fori_loop(0, nch // 2, chunk2, 0)
    for b2 in range(2):
        pltpu.make_async_copy(avs[b2], r0_out.at[pl.ds(0, CH4)],
                              semws[b2]).wait()
        pltpu.make_async_copy(cvs[b2], r1_out.at[pl.ds(0, CH4)],
                              semws[b2]).wait()


def _k4_call(st, dt, pq):
    o = jax.ShapeDtypeStruct((ET, LH), _f32)
    f = pl.kernel(
        _k4_body,
        out_type=(o, o),
        mesh=_mesh(),
        scratch_types=(
            [pltpu.VMEM((CH4,), _i32)] * 8
            + [pltpu.VMEM((CH4, LH), _f32)] * 8
            + [pltpu.SemaphoreType.DMA] * 4
        ),
    )
    return f(st, dt, pq)


RB = 256
NRB = NP // RB


def _m1_body(xp, wg, ats, atd, xlbm, als128, ald128):
    a = jnp.dot(xp[...], wg[...], preferred_element_type=_f32)
    a3 = a.reshape(RB, H, HID)
    xlbm[...] = a3.swapaxes(0, 1)
    als = (a3 * ats[...][None]).sum(-1)
    ald = (a3 * atd[...][None]).sum(-1)
    z = jnp.zeros((RB, HID - H), _f32)
    als128[...] = jnp.concatenate([als, z], axis=1)
    ald128[...] = jnp.concatenate([ald, z], axis=1)


def _m1_call(xp, W_gat, att_src, att_dst):
    return pl.pallas_call(
        _m1_body,
        grid=(NRB,),
        in_specs=[
            pl.BlockSpec((RB, IN), lambda i: (i, 0)),
            pl.BlockSpec((IN, H * HID), lambda i: (0, 0)),
            pl.BlockSpec((H, HID), lambda i: (0, 0)),
            pl.BlockSpec((H, HID), lambda i: (0, 0)),
        ],
        out_specs=[
            pl.BlockSpec((H, RB, HID), lambda i: (0, i, 0)),
            pl.BlockSpec((RB, HID), lambda i: (i, 0)),
            pl.BlockSpec((RB, HID), lambda i: (i, 0)),
        ],
        out_shape=[
            jax.ShapeDtypeStruct((H, NP, HID), _f32),
            jax.ShapeDtypeStruct((NP, HID), _f32),
            jax.ShapeDtypeStruct((NP, HID), _f32),
        ],
    )(xp, W_gat, att_src, att_dst)


def _m2b_body(degp, dinv128):
    d = degp[0] + degp[1] + 1.0
    dinv128[...] = lax.rsqrt(d)


def _m2b_call(degp):
    return pl.pallas_call(
        _m2b_body,
        grid=(NRB,),
        in_specs=[pl.BlockSpec((NC, RB, HID), lambda i: (0, i, 0))],
        out_specs=pl.BlockSpec((RB, HID), lambda i: (i, 0)),
        out_shape=jax.ShapeDtypeStruct((NP, HID), _f32),
    )(degp)


def _rowmask(i):
    rows = i * RB + lax.broadcasted_iota(_i32, (RB, 1), 0)
    return rows < N


def _m2_body(agg, sp, bg, g, b, out, acc):
    p = pl.program_id(0)
    i = pl.program_id(1)
    s = sp[0] + sp[1]
    s4 = s[:, :H]
    at = agg[...].swapaxes(0, 1)
    y3 = at / (s4[:, :, None] + 1e-16)
    y = y3.reshape(RB, H * HID) + bg[0]

    @pl.when(p == 0)
    def _():
        @pl.when(i == 0)
        def _():
            acc[...] = jnp.zeros_like(acc)
        acc[0] = acc[0] + y.sum(0)
        acc[1] = acc[1] + (y * y).sum(0)

    @pl.when(p == 1)
    def _():
        mean = acc[0] / N
        var = acc[1] / N - mean * mean
        yn = (y - mean) * lax.rsqrt(var + 1e-5) * g[0] + b[0]
        out[...] = jnp.where(_rowmask(i), jnp.maximum(yn, 0.0), 0.0)


def _m2_call(agg, sparts, b_gat, bn1_g, bn1_b):
    D = H * HID
    return pl.pallas_call(
        _m2_body,
        grid=(2, NRB),
        in_specs=[
            pl.BlockSpec((H, RB, HID), lambda p, i: (0, i, 0)),
            pl.BlockSpec((NC, RB, HID), lambda p, i: (0, i, 0)),
            pl.BlockSpec((1, D), lambda p, i: (0, 0)),
            pl.BlockSpec((1, D), lambda p, i: (0, 0)),
            pl.BlockSpec((1, D), lambda p, i: (0, 0)),
        ],
        out_specs=pl.BlockSpec((RB, D), lambda p, i: (i, 0)),
        out_shape=jax.ShapeDtypeStruct((NP, D), _f32),
        scratch_shapes=[pltpu.VMEM((8, D), _f32)],
    )(agg, sparts, b_gat, bn1_g, bn1_b)


def _mm_body(h, w, out):
    out[...] = jnp.dot(h[...], w[0], preferred_element_type=_f32)[None]


def _mm_call(h, wstack, nb, din):
    return pl.pallas_call(
        _mm_body,
        grid=(nb, NRB),
        in_specs=[
            pl.BlockSpec((RB, din), lambda j, r: (r, 0)),
            pl.BlockSpec((1, din, HID), lambda j, r: (j, 0, 0)),
        ],
        out_specs=pl.BlockSpec((1, RB, HID), lambda j, r: (j, r, 0)),
        out_shape=jax.ShapeDtypeStruct((nb, NP, HID), _f32),
    )(h, wstack)


def _m4_body(agg, hp, dinv, bc, g, b, out, acc):
    p = pl.program_id(0)
    i = pl.program_id(1)
    dv = dinv[:, :NE]
    a6 = agg[...].swapaxes(0, 1)
    at = jnp.concatenate([a6[:, :NE - 1],
                          (a6[:, NE - 1:NE] + a6[:, NE:NE + 1])], axis=1)
    ht = hp[...].swapaxes(0, 1)
    y3 = at * dv[:, :, None] + ht * (dv * dv)[:, :, None] + bc[...][None]
    y = y3.reshape(RB, NE * HID)

    @pl.when(p == 0)
    def _():
        @pl.when(i == 0)
        def _():
            acc[...] = jnp.zeros_like(acc)
        acc[0] = acc[0] + y.sum(0)
        acc[1] = acc[1] + (y * y).sum(0)

    @pl.when(p == 1)
    def _():
        mean = acc[0] / N
        var = acc[1] / N - mean * mean
        yn = (y - mean) * lax.rsqrt(var + 1e-5) * g[0] + b[0]
        out[...] = jnp.where(_rowmask(i), jnp.maximum(yn, 0.0), 0.0)


def _m4_call(agg, hp, dinv16, b_c, bn2_g, bn2_b):
    D = NE * HID
    return pl.pallas_call(
        _m4_body,
        grid=(2, NRB),
        in_specs=[
            pl.BlockSpec((NE + 1, RB, HID), lambda p, i: (0, i, 0)),
            pl.BlockSpec((NE, RB, HID), lambda p, i: (0, i, 0)),
            pl.BlockSpec((RB, HID), lambda p, i: (i, 0)),
            pl.BlockSpec((NE, HID), lambda p, i: (0, 0)),
            pl.BlockSpec((1, D), lambda p, i: (0, 0)),
            pl.BlockSpec((1, D), lambda p, i: (0, 0)),
        ],
        out_specs=pl.BlockSpec((RB, D), lambda p, i: (i, 0)),
        out_shape=jax.ShapeDtypeStruct((NP, D), _f32),
        scratch_shapes=[pltpu.VMEM((8, D), _f32)],
    )(agg, hp, dinv16, b_c, bn2_g, bn2_b)


def _m6_body(agg, hp, dinv, bml, g, b, omu, ols, acc):
    p = pl.program_id(0)
    i = pl.program_id(1)
    dv = dinv[:, :NE]
    dvv = jnp.concatenate([dv, dv], axis=1)
    at = agg[...].swapaxes(0, 1)
    ht = hp[...].swapaxes(0, 1)
    y3 = at * dvv[:, :, None] + ht * (dvv * dvv)[:, :, None] + bml[...][None]
    D = NE * HID
    ymu = y3[:, :NE].reshape(RB, D)
    yls = y3[:, NE:].reshape(RB, D)

    @pl.when(p == 0)
    def _():
        @pl.when(i == 0)
        def _():
            acc[...] = jnp.zeros_like(acc)
        acc[0] = acc[0] + ymu.sum(0)
        acc[1] = acc[1] + (ymu * ymu).sum(0)
        acc[2] = acc[2] + yls.sum(0)
        acc[3] = acc[3] + (yls * yls).sum(0)

    @pl.when(p == 1)
    def _():
        m = _rowmask(i)
        mu_m = acc[0] / N
        mu_v = acc[1] / N - mu_m * mu_m
        ls_m = acc[2] / N
        ls_v = acc[3] / N - ls_m * ls_m
        a = (ymu - mu_m) * lax.rsqrt(mu_v + 1e-5) * g[0] + b[0]
        c = (yls - ls_m) * lax.rsqrt(ls_v + 1e-5) * g[0] + b[0]
        omu[...] = jnp.where(m, jnp.maximum(a, 0.0), 0.0)
        ols[...] = jnp.where(m, jnp.maximum(c, 0.0), 0.0)


def _m6_call(agg, hp, dinv16, b_ml, bn2_g, bn2_b):
    D = NE * HID
    return pl.pallas_call(
        _m6_body,
        grid=(2, NRB),
        in_specs=[
            pl.BlockSpec((2 * NE, RB, HID), lambda p, i: (0, i, 0)),
            pl.BlockSpec((2 * NE, RB, HID), lambda p, i: (0, i, 0)),
            pl.BlockSpec((RB, HID), lambda p, i: (i, 0)),
            pl.BlockSpec((2 * NE, HID), lambda p, i: (0, 0)),
            pl.BlockSpec((1, D), lambda p, i: (0, 0)),
            pl.BlockSpec((1, D), lambda p, i: (0, 0)),
        ],
        out_specs=[
            pl.BlockSpec((RB, D), lambda p, i: (i, 0)),
            pl.BlockSpec((RB, D), lambda p, i: (i, 0)),
        ],
        out_shape=[
            jax.ShapeDtypeStruct((NP, D), _f32),
            jax.ShapeDtypeStruct((NP, D), _f32),
        ],
        scratch_shapes=[pltpu.VMEM((8, D), _f32)],
    )(agg, hp, dinv16, b_ml, bn2_g, bn2_b)


EB = 512
NEB = ET // EB


def _m8_body(r0, r1, b0, w1, b1, w2, b2, out):
    wb = w1[...]
    z0 = jnp.maximum(r0[...] + b0[0], 0.0).astype(jnp.bfloat16)
    z1 = jnp.maximum(r1[...] + b0[0], 0.0).astype(jnp.bfloat16)
    t0 = jnp.maximum(jnp.dot(z0, wb, preferred_element_type=_f32) + b1[0], 0.0)
    t1 = jnp.maximum(jnp.dot(z1, wb, preferred_element_type=_f32) + b1[0], 0.0)
    out[...] = jnp.dot(0.5 * (t0 + t1), w2[...], preferred_element_type=_f32) + b2[0]


def _m8_call(r0, r1, b_l0, W_l1, b_l1, W_l2p, b_l2p):
    es = pl.BlockSpec((EB, LH), lambda i: (i, 0))
    return pl.pallas_call(
        _m8_body,
        grid=(NEB,),
        in_specs=[
            es, es,
            pl.BlockSpec((1, LH), lambda i: (0, 0)),
            pl.BlockSpec((LH, LH), lambda i: (0, 0)),
            pl.BlockSpec((1, LH), lambda i: (0, 0)),
            pl.BlockSpec((LH, HID), lambda i: (0, 0)),
            pl.BlockSpec((1, HID), lambda i: (0, 0)),
        ],
        out_specs=pl.BlockSpec((EB, HID), lambda i: (i, 0)),
        out_shape=jax.ShapeDtypeStruct((ET, HID), _f32),
    )(r0, r1, b_l0, W_l1, b_l1, W_l2p, b_l2p)


def _pad_idx(n_extra):
    return (N + (jnp.arange(n_extra, dtype=_i32) % (NP - N))).astype(_i32)


def kernel(x, edge_index, edge_attr, edge_index_test, W_gat, att_src, att_dst,
           b_gat, bn1_g, bn1_b, bn2_g, bn2_b, W_c, b_c, W_mu, b_mu, W_ls, b_ls,
           W_l0, b_l0, W_l1, b_l1, W_l2, b_l2):
    xp = jnp.pad(x, ((0, NP - N), (0, 0)))
    loop = jnp.arange(N, dtype=_i32)
    srcg = jnp.concatenate([edge_index[0], loop, _pad_idx(EG - E - N)])
    dstg = jnp.concatenate([edge_index[1], loop, _pad_idx(EG - E - N)])
    srcw = jnp.concatenate([edge_index[0], _pad_idx(EW - E)])
    dstw = jnp.concatenate([edge_index[1], _pad_idx(EW - E)])
    st = jnp.concatenate([edge_index_test[0], _pad_idx(ET - E)])
    dt = jnp.concatenate([edge_index_test[1], _pad_idx(ET - E)])
    wflat = jnp.pad(edge_attr, ((0, EW - E), (0, L - NE))).reshape(EW * L)
    z128 = jnp.zeros((NP, HID), _f32)

    W_mlp = jnp.concatenate([W_mu, W_ls], axis=0)
    b_mlp = jnp.concatenate([b_mu, b_ls], axis=0)
    W0ab = W_l0.reshape(2, NE * HID, LH)
    W_l2p = jnp.pad(W_l2, ((0, 0), (0, HID - 4)))
    b_l2p = jnp.pad(b_l2, (0, HID - 4)).reshape(1, HID)
    b_gat2 = b_gat.reshape(1, -1)
    bn1_g2, bn1_b2 = bn1_g.reshape(1, -1), bn1_b.reshape(1, -1)
    bn2_g2, bn2_b2 = bn2_g.reshape(1, -1), bn2_b.reshape(1, -1)
    b_l0_2, b_l1_2 = b_l0.reshape(1, -1), b_l1.reshape(1, -1)

    xlbm, als128, ald128 = _m1_call(xp, W_gat, att_src, att_dst)
    e_sc = _k1a_call(srcg, dstg, als128, ald128)
    sparts = _k2a_call(dstg, e_sc, z128, EG)
    agg1 = _k3_call(srcg, dstg, e_sc, xlbm.reshape(H * NP, HID), z128,
                    nbp=H, ncol=16, etot=EG)
    h1 = _m2_call(agg1.reshape(H, NP, HID), sparts.reshape(NC, NP, HID),
                  b_gat2, bn1_g2, bn1_b2)

    degp = _k2a_call(dstw, wflat, z128, EW)
    dinv128 = _m2b_call(degp.reshape(NC, NP, HID))
    scale = _k2b_call(srcw, wflat, dinv128)

    hp_c = _mm_call(h1, W_c, NE, H * HID)
    agg2 = _k3_call(srcw, dstw, scale, hp_c.reshape(NE * NP, HID), z128,
                    nbp=NE, ncol=16, etot=EW)
    h2 = _m4_call(agg2.reshape(NE + 1, NP, HID), hp_c, dinv128, b_c,
                  bn2_g2, bn2_b2)

    hp_ml = _mm_call(h2, W_mlp, 2 * NE, NE * HID)
    agg3 = _k3_call(srcw, dstw, scale, hp_ml.reshape(2 * NE * NP, HID), z128,
                    nbp=2 * NE, ncol=NE, etot=EW)
    xmu_p, xls_p = _m6_call(agg3.reshape(2 * NE, NP, HID), hp_ml, dinv128,
                            b_mlp, bn2_g2, bn2_b2)

    pq = _mm_call2(xmu_p, W0ab)
    r0, r1 = _k4_call(st, dt, pq.reshape(2 * NP, LH))
    dfull = _m8_call(r0, r1, b_l0_2, W_l1.astype(jnp.bfloat16),
                     b_l1_2, W_l2p, b_l2p)

    return (xmu_p[:N], xls_p[:N], dfull[:E, :4])


def _mm2_body(h, w, out):
    out[...] = jnp.dot(h[...], w[0], preferred_element_type=_f32)[None]


def _mm_call2(h, wstack):
    return pl.pallas_call(
        _mm2_body,
        grid=(2, NRB),
        in_specs=[
            pl.BlockSpec((RB, NE * HID), lambda j, r: (r, 0)),
            pl.BlockSpec((1, NE * HID, LH), lambda j, r: (j, 0, 0)),
        ],
        out_specs=pl.BlockSpec((1, RB, LH), lambda j, r: (j, r, 0)),
        out_shape=jax.ShapeDtypeStruct((2, NP, LH), _f32),
    )(h, wstack)
